# Initial kernel scaffold; baseline (speedup 1.0000x reference)
import jax
import jax.numpy as jnp
from jax.experimental import pallas as pl


def kernel(edge_vec, node_feature, edge_index, edge_feature, lin1_w, lin1_b, fc1_w1, fc1_b1, fc1_w2, fc1_b2, fc2_w1, fc2_b1, fc2_w2, fc2_b2, bn_gamma, bn_beta, lin2_w, lin2_b):
    raise NotImplementedError("write your pallas kernel here")


# speedup vs baseline: 1.8849x; 1.8849x over previous
import functools

import jax
import jax.numpy as jnp
import numpy as np
from jax import lax
from jax.experimental import pallas as pl
from jax.experimental.pallas import tpu as pltpu
from jax.experimental.pallas import tpu_sc as plsc

N_NODES = 10000
N_EDGES = 160000
NW = 32
GRP = 128
GBLK = 8
CH = GRP * GBLK
E_PAD = 163840
EB = 2048
S3 = float(np.sqrt(3.0))
S5 = float(np.sqrt(5.0))
S15 = float(np.sqrt(15.0))
C1 = 1.0 / float(np.sqrt(16.0))
C2 = 1.0 / float(np.sqrt(24.0))


def _softplus(x):
    m = jnp.maximum(x, 0.0)
    return m + jnp.log(jnp.exp(x - m) + jnp.exp(-m))


def _np_consts():
    col1 = np.zeros((16, 24), dtype=np.int32)
    for u in range(16):
        for j in range(24):
            if j < 16:
                col1[u, j] = u * 16 + j
            elif j < 20:
                col1[u, j] = 256 + u * 4 + (j - 16)
            else:
                col1[u, j] = 320 + u * 4 + (j - 20)
    col2 = np.zeros((24, 16), dtype=np.int32)
    for u in range(24):
        for w in range(16):
            if u < 16:
                col2[u, w] = u * 16 + w
            elif u < 20:
                col2[u, w] = 256 + (u - 16) * 16 + w
            else:
                col2[u, w] = 320 + (u - 20) * 16 + w
    RK = np.repeat(np.eye(16, dtype=np.float32), 16, axis=1)
    TU = np.tile(np.eye(16, dtype=np.float32), (1, 16))
    RK2 = np.repeat(np.eye(16, dtype=np.float32), 24, axis=1)
    TU2 = np.tile(np.eye(24, dtype=np.float32), (1, 16))
    E1 = np.kron(np.eye(4, dtype=np.float32), np.ones((1, 3), np.float32))
    E2 = np.kron(np.eye(4, dtype=np.float32), np.ones((1, 5), np.float32))
    P1 = np.kron(np.eye(4, dtype=np.float32), np.ones((3, 1), np.float32))
    P2 = np.kron(np.eye(4, dtype=np.float32), np.ones((5, 1), np.float32))
    return col1, col2, RK, TU, RK2, TU2, E1, E2, P1, P2


_COL1, _COL2, _RK, _TU, _RK2, _TU2, _E1, _E2, _P1, _P2 = _np_consts()


def _sc_gather(table, idx2d, d):
    epw = E_PAD // NW
    gpw = epw // CH
    mesh = plsc.VectorSubcoreMesh(core_axis_name="c", subcore_axis_name="s")

    @functools.partial(
        pl.kernel, mesh=mesh,
        out_type=jax.ShapeDtypeStruct((E_PAD, d), jnp.float32),
        compiler_params=pltpu.CompilerParams(use_tc_tiling_on_sc=False),
        scratch_types=[
            pltpu.VMEM((GBLK, GRP), jnp.int32),
            pltpu.VMEM((CH, d), jnp.float32),
            pltpu.SemaphoreType.DMA,
        ],
    )
    def k(table_hbm, idx_hbm, out_hbm, idx_v, rows_v, sem):
        wid = lax.axis_index("s") * 2 + lax.axis_index("c")
        base_g = wid * (epw // GRP)

        def body(i, carry):
            goff = base_g + i * GBLK
            eoff = goff * GRP
            pltpu.sync_copy(idx_hbm.at[pl.ds(goff, GBLK)], idx_v)
            for j in range(GBLK):
                pltpu.async_copy(
                    table_hbm.at[idx_v.at[j]],
                    rows_v.at[pl.ds(j * GRP, GRP)], sem).wait()
            pltpu.sync_copy(rows_v, out_hbm.at[pl.ds(eoff, CH)])
            return carry

        lax.fori_loop(0, gpw, body, 0)

    return k(table, idx2d)


def _sc_scatter_add(vals, idx2d, d):
    epw = E_PAD // NW
    gpw = epw // CH
    rows_pt = N_NODES // 16
    mesh = plsc.VectorSubcoreMesh(core_axis_name="c", subcore_axis_name="s")
    zeros = jnp.zeros((N_NODES, d), jnp.float32)

    @functools.partial(
        pl.kernel, mesh=mesh,
        out_type=jax.ShapeDtypeStruct((2, N_NODES, d), jnp.float32),
        compiler_params=pltpu.CompilerParams(use_tc_tiling_on_sc=False),
        scratch_types=[
            pltpu.VMEM((GBLK, GRP), jnp.int32),
            pltpu.VMEM((CH, d), jnp.float32),
            pltpu.VMEM_SHARED((N_NODES, d), jnp.float32),
            pltpu.SemaphoreType.DMA,
        ],
    )
    def k(vals_hbm, idx_hbm, zero_hbm, out_hbm, idx_v, rows_v, acc_sh, sem):
        cid = lax.axis_index("c")
        sid = lax.axis_index("s")
        wid = sid * 2 + cid
        r0 = sid * rows_pt
        pltpu.sync_copy(zero_hbm.at[pl.ds(r0, rows_pt)],
                        acc_sh.at[pl.ds(r0, rows_pt)])
        plsc.subcore_barrier()
        base_g = wid * (epw // GRP)

        def body(i, carry):
            goff = base_g + i * GBLK
            eoff = goff * GRP
            pltpu.sync_copy(idx_hbm.at[pl.ds(goff, GBLK)], idx_v)
            pltpu.sync_copy(vals_hbm.at[pl.ds(eoff, CH)], rows_v)
            for j in range(GBLK):
                pltpu.sync_copy(rows_v.at[pl.ds(j * GRP, GRP)],
                                acc_sh.at[idx_v.at[j]], add=True)
            return carry

        lax.fori_loop(0, gpw, body, 0)
        plsc.subcore_barrier()
        pltpu.sync_copy(acc_sh.at[pl.ds(r0, rows_pt)],
                        out_hbm.at[cid, pl.ds(r0, rows_pt)])

    return k(vals, idx2d, zeros)


def _tc_h(node_feature, lin1_w, lin1_b):
    def body(nf, w, b, o):
        o[...] = jnp.dot(nf[...], w[...],
                         preferred_element_type=jnp.float32) + b[...]

    return pl.pallas_call(
        body,
        out_shape=jax.ShapeDtypeStruct((N_NODES, 16), jnp.float32),
    )(node_feature, lin1_w, lin1_b.reshape(1, 16))


def _sh_from_vec(ev):
    nrm = jnp.sqrt(jnp.sum(ev * ev, axis=1, keepdims=True))
    v = ev / jnp.where(nrm == 0.0, 1.0, nrm)
    x, y, z = v[:, 0:1], v[:, 1:2], v[:, 2:3]
    sh1 = jnp.concatenate([S3 * x, S3 * y, S3 * z], axis=1)
    sh2 = jnp.concatenate([
        S15 * x * z, S15 * x * y,
        S5 * (y * y - 0.5 * (x * x + z * z)),
        S15 * y * z, 0.5 * S15 * (z * z - x * x)], axis=1)
    return sh1, sh2


def _tc_tp1(ev_p, ef_p, hd, fc1_w1, fc1_b1, A1, B1):
    grid = E_PAD // EB

    def body(ev_ref, ef_ref, hd_ref, w1_ref, b1_ref, a_ref, bb_ref,
             rk_ref, tu_ref, e1_ref, e2_ref, o_ref):
        i = pl.program_id(0)
        sh1, sh2 = _sh_from_vec(ev_ref[...])
        sh1t = jnp.concatenate([sh1] * 4, axis=1)
        sh2t = jnp.concatenate([sh2] * 4, axis=1)
        f1 = _softplus(jnp.dot(ef_ref[...], w1_ref[...],
                               preferred_element_type=jnp.float32) + b1_ref[...])
        hd_b = hd_ref[...]
        g = (jnp.dot(f1, rk_ref[...], preferred_element_type=jnp.float32)
             * jnp.dot(hd_b, tu_ref[...], preferred_element_type=jnp.float32))
        t = C1 * (jnp.dot(g, a_ref[...], preferred_element_type=jnp.float32)
                  + jnp.dot(hd_b, bb_ref[...], preferred_element_type=jnp.float32))
        t1e = jnp.dot(t[:, 16:20], e1_ref[...],
                      preferred_element_type=jnp.float32) * sh1t
        t2e = jnp.dot(t[:, 20:24], e2_ref[...],
                      preferred_element_type=jnp.float32) * sh2t
        rows = i * EB + lax.broadcasted_iota(jnp.int32, (EB, 1), 0)
        mask = (rows < N_EDGES).astype(jnp.float32)
        out = jnp.concatenate(
            [t[:, :16], t1e, t2e, jnp.ones((EB, 1), jnp.float32),
             jnp.zeros((EB, 15), jnp.float32)], axis=1)
        o_ref[...] = out * mask

    full = lambda shape: pl.BlockSpec(shape, lambda i: (0, 0))
    blk = lambda d: pl.BlockSpec((EB, d), lambda i: (i, 0))
    return pl.pallas_call(
        body,
        grid=(grid,),
        in_specs=[blk(3), blk(16), blk(16), full((16, 16)), full((1, 16)),
                  full((256, 24)), full((16, 24)), full((16, 256)),
                  full((16, 256)), full((4, 12)), full((4, 20))],
        out_specs=blk(64),
        out_shape=jax.ShapeDtypeStruct((E_PAD, 64), jnp.float32),
    )(ev_p, ef_p, hd, fc1_w1, fc1_b1.reshape(1, 16), A1, B1,
      jnp.asarray(_RK), jnp.asarray(_TU), jnp.asarray(_E1), jnp.asarray(_E2))


def _tc_out1(p0, p1, h):
    def body(p0_ref, p1_ref, h_ref, o_ref, c_ref):
        s = p0_ref[...] + p1_ref[...]
        cnt = jnp.maximum(s[:, 48:49], 1.0)
        o = s[:, :48] / cnt
        o = o + jnp.concatenate(
            [h_ref[...], jnp.zeros((N_NODES, 32), jnp.float32)], axis=1)
        o_ref[...] = o
        c_ref[...] = cnt

    return pl.pallas_call(
        body,
        out_shape=(jax.ShapeDtypeStruct((N_NODES, 48), jnp.float32),
                   jax.ShapeDtypeStruct((N_NODES, 1), jnp.float32)),
    )(p0, p1, h)


def _tc_tp2(ev_p, ef_p, od, fc2_w1, fc2_b1, A2, B2):
    grid = E_PAD // EB

    def body(ev_ref, ef_ref, od_ref, w1_ref, b1_ref, a_ref, bb_ref,
             rk_ref, tu_ref, p1_ref, p2_ref, o_ref):
        i = pl.program_id(0)
        sh1, sh2 = _sh_from_vec(ev_ref[...])
        sh1t = jnp.concatenate([sh1] * 4, axis=1)
        sh2t = jnp.concatenate([sh2] * 4, axis=1)
        f2 = _softplus(jnp.dot(ef_ref[...], w1_ref[...],
                               preferred_element_type=jnp.float32) + b1_ref[...])
        od_b = od_ref[...]
        d1 = jnp.dot(od_b[:, 16:28] * sh1t, p1_ref[...],
                     preferred_element_type=jnp.float32) * (1.0 / S3)
        d2 = jnp.dot(od_b[:, 28:48] * sh2t, p2_ref[...],
                     preferred_element_type=jnp.float32) * (1.0 / S5)
        yv = jnp.concatenate([od_b[:, :16], d1, d2], axis=1)
        g = (jnp.dot(f2, rk_ref[...], preferred_element_type=jnp.float32)
             * jnp.dot(yv, tu_ref[...], preferred_element_type=jnp.float32))
        t = C2 * (jnp.dot(g, a_ref[...], preferred_element_type=jnp.float32)
                  + jnp.dot(yv, bb_ref[...], preferred_element_type=jnp.float32))
        rows = i * EB + lax.broadcasted_iota(jnp.int32, (EB, 1), 0)
        mask = (rows < N_EDGES).astype(jnp.float32)
        o_ref[...] = t * mask

    full = lambda shape: pl.BlockSpec(shape, lambda i: (0, 0))
    blk = lambda d: pl.BlockSpec((EB, d), lambda i: (i, 0))
    return pl.pallas_call(
        body,
        grid=(grid,),
        in_specs=[blk(3), blk(16), blk(48), full((16, 16)), full((1, 16)),
                  full((384, 16)), full((24, 16)), full((16, 384)),
                  full((24, 384)), full((12, 4)), full((20, 4))],
        out_specs=blk(16),
        out_shape=jax.ShapeDtypeStruct((E_PAD, 16), jnp.float32),
    )(ev_p, ef_p, od, fc2_w1, fc2_b1.reshape(1, 16), A2, B2,
      jnp.asarray(_RK2), jnp.asarray(_TU2), jnp.asarray(_P1), jnp.asarray(_P2))


def _tc_final(q0, q1, cntc, bn_gamma, bn_beta, lin2_w, lin2_b, skip):
    def body(q0_ref, q1_ref, c_ref, g_ref, b_ref, w_ref, wb_ref, s_ref, o_ref):
        o2 = (q0_ref[...] + q1_ref[...]) / c_ref[...]
        mean = jnp.mean(o2, axis=0, keepdims=True)
        var = jnp.mean((o2 - mean) ** 2, axis=0, keepdims=True)
        xn = (o2 - mean) / jnp.sqrt(var + 1e-5) * g_ref[...] + b_ref[...]
        h3 = _softplus(jnp.dot(_softplus(xn), w_ref[...],
                               preferred_element_type=jnp.float32) + wb_ref[...])
        o_ref[...] = h3 + s_ref[...]

    return pl.pallas_call(
        body,
        out_shape=jax.ShapeDtypeStruct((N_NODES, 128), jnp.float32),
    )(q0, q1, cntc, bn_gamma.reshape(1, 16), bn_beta.reshape(1, 16),
      lin2_w, lin2_b.reshape(1, 128), skip)


def kernel(edge_vec, node_feature, edge_index, edge_feature, lin1_w, lin1_b,
           fc1_w1, fc1_b1, fc1_w2, fc1_b2, fc2_w1, fc2_b1, fc2_w2, fc2_b2,
           bn_gamma, bn_beta, lin2_w, lin2_b):
    A1 = fc1_w2[:, _COL1].reshape(256, 24)
    B1 = fc1_b2[_COL1]
    A2 = fc2_w2[:, _COL2].reshape(384, 16)
    B2 = fc2_b2[_COL2]

    pe = E_PAD - N_EDGES
    ev_p = jnp.pad(edge_vec, ((0, pe), (0, 0)), constant_values=1.0)
    ef_p = jnp.pad(edge_feature, ((0, pe), (0, 0)))
    src2d = jnp.pad(edge_index[0], (0, pe)).reshape(E_PAD // GRP, GRP)
    dst2d = jnp.pad(edge_index[1], (0, pe)).reshape(E_PAD // GRP, GRP)

    h = _tc_h(node_feature, lin1_w, lin1_b)
    hd = _sc_gather(h, dst2d, 16)
    tp1e = _tc_tp1(ev_p, ef_p, hd, fc1_w1, fc1_b1, A1, B1)
    p = _sc_scatter_add(tp1e, src2d, 64)
    out1, cntc = _tc_out1(p[0], p[1], h)
    od = _sc_gather(out1, dst2d, 48)
    tp2 = _tc_tp2(ev_p, ef_p, od, fc2_w1, fc2_b1, A2, B2)
    q = _sc_scatter_add(tp2, src2d, 16)
    return _tc_final(q[0], q[1], cntc, bn_gamma, bn_beta,
                     lin2_w, lin2_b, node_feature)


# speedup vs baseline: 2.9946x; 1.5887x over previous
import functools

import jax
import jax.numpy as jnp
import numpy as np
from jax import lax
from jax.experimental import pallas as pl
from jax.experimental.pallas import tpu as pltpu
from jax.experimental.pallas import tpu_sc as plsc

N_NODES = 10000
N_EDGES = 160000
NW = 32
GRP = 128
GBLK = 8
CH = GRP * GBLK
E_PAD = 163840
EB = 2048
S3 = float(np.sqrt(3.0))
S5 = float(np.sqrt(5.0))
S15 = float(np.sqrt(15.0))
C1 = 1.0 / float(np.sqrt(16.0))
C2 = 1.0 / float(np.sqrt(24.0))


def _softplus(x):
    m = jnp.maximum(x, 0.0)
    return m + jnp.log(jnp.exp(x - m) + jnp.exp(-m))


def _np_consts():
    col1 = np.zeros((16, 24), dtype=np.int32)
    for u in range(16):
        for j in range(24):
            if j < 16:
                col1[u, j] = u * 16 + j
            elif j < 20:
                col1[u, j] = 256 + u * 4 + (j - 16)
            else:
                col1[u, j] = 320 + u * 4 + (j - 20)
    col2 = np.zeros((24, 16), dtype=np.int32)
    for u in range(24):
        for w in range(16):
            if u < 16:
                col2[u, w] = u * 16 + w
            elif u < 20:
                col2[u, w] = 256 + (u - 16) * 16 + w
            else:
                col2[u, w] = 320 + (u - 20) * 16 + w
    RK = np.repeat(np.eye(16, dtype=np.float32), 16, axis=1)
    TU = np.tile(np.eye(16, dtype=np.float32), (1, 16))
    RK2 = np.repeat(np.eye(16, dtype=np.float32), 24, axis=1)
    TU2 = np.tile(np.eye(24, dtype=np.float32), (1, 16))
    E1 = np.kron(np.eye(4, dtype=np.float32), np.ones((1, 3), np.float32))
    E2 = np.kron(np.eye(4, dtype=np.float32), np.ones((1, 5), np.float32))
    M1 = np.zeros((3, 6), np.float32)
    M2 = np.zeros((3, 6), np.float32)
    first = [0, 0, 1, 0, 2, 1]
    second = [2, 1, 1, 0, 2, 2]
    for c in range(6):
        M1[first[c], c] = 1.0
        M2[second[c], c] = 1.0
    C2m = np.zeros((6, 5), np.float32)
    C2m[0, 0] = S15
    C2m[1, 1] = S15
    C2m[2, 2] = S5
    C2m[3, 2] = -0.5 * S5
    C2m[4, 2] = -0.5 * S5
    C2m[5, 3] = S15
    C2m[4, 4] = 0.5 * S15
    C2m[3, 4] = -0.5 * S15
    TEXP = np.zeros((24, 64), np.float32)
    for j in range(16):
        TEXP[j, j] = 1.0
    for v in range(4):
        for m in range(3):
            TEXP[16 + v, 16 + 3 * v + m] = 1.0
        for c in range(5):
            TEXP[20 + v, 28 + 5 * v + c] = 1.0
    MQ1p = np.zeros((3, 64), np.float32)
    MQ2p = np.zeros((6, 64), np.float32)
    for v in range(4):
        for m in range(3):
            MQ1p[m, 16 + 3 * v + m] = S3
        for c in range(5):
            MQ2p[:, 28 + 5 * v + c] = C2m[:, c]
    ONES64 = np.zeros((1, 64), np.float32)
    ONES64[0, :16] = 1.0
    ONES64[0, 48] = 1.0
    ONE48 = np.zeros((1, 64), np.float32)
    ONE48[0, 48] = 1.0
    MD1 = np.zeros((3, 48), np.float32)
    MD2 = np.zeros((6, 48), np.float32)
    for v in range(4):
        for m in range(3):
            MD1[m, 16 + 3 * v + m] = S3
        for c in range(5):
            MD2[:, 28 + 5 * v + c] = C2m[:, c]
    ONES16_48 = np.zeros((1, 48), np.float32)
    ONES16_48[0, :16] = 1.0
    YP = np.zeros((48, 24), np.float32)
    for j in range(16):
        YP[j, j] = 1.0
    for v in range(4):
        for m in range(3):
            YP[16 + 3 * v + m, 16 + v] = 1.0 / S3
        for c in range(5):
            YP[28 + 5 * v + c, 20 + v] = 1.0 / S5
    sh = dict(M1=M1, M2=M2, TEXP=TEXP, MQ1p=MQ1p, MQ2p=MQ2p, ONES64=ONES64,
              ONE48=ONE48, MD1=MD1, MD2=MD2, ONES16_48=ONES16_48, YP=YP,
              ONES3=np.ones((3, 1), np.float32))
    return col1, col2, RK, TU, RK2, TU2, sh


_COL1, _COL2, _RK, _TU, _RK2, _TU2, _SH = _np_consts()


def _sc_gather(table, idx2d, d):
    epw = E_PAD // NW
    gpw = epw // CH
    mesh = plsc.VectorSubcoreMesh(core_axis_name="c", subcore_axis_name="s")

    @functools.partial(
        pl.kernel, mesh=mesh,
        out_type=jax.ShapeDtypeStruct((E_PAD, d), jnp.float32),
        compiler_params=pltpu.CompilerParams(use_tc_tiling_on_sc=False),
        scratch_types=[
            pltpu.VMEM((GBLK, GRP), jnp.int32),
            pltpu.VMEM((CH, d), jnp.float32),
            pltpu.SemaphoreType.DMA,
        ],
    )
    def k(table_hbm, idx_hbm, out_hbm, idx_v, rows_v, sem):
        wid = lax.axis_index("s") * 2 + lax.axis_index("c")
        base_g = wid * (epw // GRP)

        def body(i, carry):
            goff = base_g + i * GBLK
            eoff = goff * GRP
            pltpu.sync_copy(idx_hbm.at[pl.ds(goff, GBLK)], idx_v)
            for j in range(GBLK):
                pltpu.async_copy(
                    table_hbm.at[idx_v.at[j]],
                    rows_v.at[pl.ds(j * GRP, GRP)], sem).wait()
            pltpu.sync_copy(rows_v, out_hbm.at[pl.ds(eoff, CH)])
            return carry

        lax.fori_loop(0, gpw, body, 0)

    return k(table, idx2d)


def _sc_scatter_add(vals, idx2d, d):
    epw = E_PAD // NW
    gpw = epw // CH
    rows_pt = N_NODES // 16
    mesh = plsc.VectorSubcoreMesh(core_axis_name="c", subcore_axis_name="s")
    zeros = jnp.zeros((N_NODES, d), jnp.float32)

    @functools.partial(
        pl.kernel, mesh=mesh,
        out_type=jax.ShapeDtypeStruct((2, N_NODES, d), jnp.float32),
        compiler_params=pltpu.CompilerParams(use_tc_tiling_on_sc=False),
        scratch_types=[
            pltpu.VMEM((GBLK, GRP), jnp.int32),
            pltpu.VMEM((CH, d), jnp.float32),
            pltpu.VMEM_SHARED((N_NODES, d), jnp.float32),
            pltpu.SemaphoreType.DMA,
        ],
    )
    def k(vals_hbm, idx_hbm, zero_hbm, out_hbm, idx_v, rows_v, acc_sh, sem):
        cid = lax.axis_index("c")
        sid = lax.axis_index("s")
        wid = sid * 2 + cid
        r0 = sid * rows_pt
        pltpu.sync_copy(zero_hbm.at[pl.ds(r0, rows_pt)],
                        acc_sh.at[pl.ds(r0, rows_pt)])
        plsc.subcore_barrier()
        base_g = wid * (epw // GRP)

        def body(i, carry):
            goff = base_g + i * GBLK
            eoff = goff * GRP
            pltpu.sync_copy(idx_hbm.at[pl.ds(goff, GBLK)], idx_v)
            pltpu.sync_copy(vals_hbm.at[pl.ds(eoff, CH)], rows_v)
            for j in range(GBLK):
                pltpu.sync_copy(rows_v.at[pl.ds(j * GRP, GRP)],
                                acc_sh.at[idx_v.at[j]], add=True)
            return carry

        lax.fori_loop(0, gpw, body, 0)
        plsc.subcore_barrier()
        pltpu.sync_copy(acc_sh.at[pl.ds(r0, rows_pt)],
                        out_hbm.at[cid, pl.ds(r0, rows_pt)])

    return k(vals, idx2d, zeros)


def _tc_h(node_feature, lin1_w, lin1_b):
    def body(nf, w, b, o):
        o[...] = jnp.dot(nf[...], w[...],
                         preferred_element_type=jnp.float32) + b[...]

    return pl.pallas_call(
        body,
        out_shape=jax.ShapeDtypeStruct((N_NODES, 16), jnp.float32),
    )(node_feature, lin1_w, lin1_b.reshape(1, 16))


def _dot(a, b):
    return jnp.dot(a, b, preferred_element_type=jnp.float32)


def _sh_expand(ev, mq1, mq2, ones_row, m1, m2, ones3):
    ss = _dot(ev * ev, ones3)
    rr = jnp.where(ss == 0.0, 1.0, ss)
    rinv = lax.rsqrt(rr)
    r2inv = 1.0 / rr
    evn = ev * rinv
    praw = (_dot(ev, m1) * _dot(ev, m2)) * r2inv
    return ones_row + _dot(evn, mq1) + _dot(praw, mq2)


def _tc_tp1(ev_p, ef_p, hd, fc1_w1, fc1_b1, A1, B1):
    grid = E_PAD // EB

    def body(ev_ref, ef_ref, hd_ref, w1_ref, b1_ref, a_ref, bb_ref,
             rk_ref, tu_ref, m1_ref, m2_ref, ones3_ref, mq1_ref, mq2_ref,
             ones64_ref, one48_ref, texp_ref, o_ref):
        i = pl.program_id(0)
        shE = _sh_expand(ev_ref[...], mq1_ref[...], mq2_ref[...],
                         ones64_ref[...], m1_ref[...], m2_ref[...],
                         ones3_ref[...])
        f1 = _softplus(_dot(ef_ref[...], w1_ref[...]) + b1_ref[...])
        hd_b = hd_ref[...]
        g = _dot(f1, rk_ref[...]) * _dot(hd_b, tu_ref[...])
        t = C1 * (_dot(g, a_ref[...]) + _dot(hd_b, bb_ref[...]))
        rows = i * EB + lax.broadcasted_iota(jnp.int32, (EB, 1), 0)
        mask = (rows < N_EDGES).astype(jnp.float32)
        o_ref[...] = (_dot(t, texp_ref[...]) + one48_ref[...]) * shE * mask

    full = lambda shape: pl.BlockSpec(shape, lambda i: (0, 0))
    blk = lambda d: pl.BlockSpec((EB, d), lambda i: (i, 0))
    return pl.pallas_call(
        body,
        grid=(grid,),
        in_specs=[blk(3), blk(16), blk(16), full((16, 16)), full((1, 16)),
                  full((256, 24)), full((16, 24)), full((16, 256)),
                  full((16, 256)), full((3, 6)), full((3, 6)), full((3, 1)),
                  full((3, 64)), full((6, 64)), full((1, 64)), full((1, 64)),
                  full((24, 64))],
        out_specs=blk(64),
        out_shape=jax.ShapeDtypeStruct((E_PAD, 64), jnp.float32),
    )(ev_p, ef_p, hd, fc1_w1, fc1_b1.reshape(1, 16), A1, B1,
      jnp.asarray(_RK), jnp.asarray(_TU),
      jnp.asarray(_SH['M1']), jnp.asarray(_SH['M2']), jnp.asarray(_SH['ONES3']),
      jnp.asarray(_SH['MQ1p']), jnp.asarray(_SH['MQ2p']),
      jnp.asarray(_SH['ONES64']), jnp.asarray(_SH['ONE48']),
      jnp.asarray(_SH['TEXP']))


def _tc_out1(p0, p1, h):
    def body(p0_ref, p1_ref, h_ref, o_ref, c_ref):
        s = p0_ref[...] + p1_ref[...]
        cnt = jnp.maximum(s[:, 48:49], 1.0)
        o = s[:, :48] / cnt
        o = o + jnp.concatenate(
            [h_ref[...], jnp.zeros((N_NODES, 32), jnp.float32)], axis=1)
        o_ref[...] = o
        c_ref[...] = cnt

    return pl.pallas_call(
        body,
        out_shape=(jax.ShapeDtypeStruct((N_NODES, 48), jnp.float32),
                   jax.ShapeDtypeStruct((N_NODES, 1), jnp.float32)),
    )(p0, p1, h)


def _tc_tp2(ev_p, ef_p, od, fc2_w1, fc2_b1, A2, B2):
    grid = E_PAD // EB

    def body(ev_ref, ef_ref, od_ref, w1_ref, b1_ref, a_ref, bb_ref,
             rk_ref, tu_ref, m1_ref, m2_ref, ones3_ref, md1_ref, md2_ref,
             ones48_ref, yp_ref, o_ref):
        i = pl.program_id(0)
        shD = _sh_expand(ev_ref[...], md1_ref[...], md2_ref[...],
                         ones48_ref[...], m1_ref[...], m2_ref[...],
                         ones3_ref[...])
        f2 = _softplus(_dot(ef_ref[...], w1_ref[...]) + b1_ref[...])
        yv = _dot(od_ref[...] * shD, yp_ref[...])
        g = _dot(f2, rk_ref[...]) * _dot(yv, tu_ref[...])
        t = C2 * (_dot(g, a_ref[...]) + _dot(yv, bb_ref[...]))
        rows = i * EB + lax.broadcasted_iota(jnp.int32, (EB, 1), 0)
        mask = (rows < N_EDGES).astype(jnp.float32)
        o_ref[...] = t * mask

    full = lambda shape: pl.BlockSpec(shape, lambda i: (0, 0))
    blk = lambda d: pl.BlockSpec((EB, d), lambda i: (i, 0))
    return pl.pallas_call(
        body,
        grid=(grid,),
        in_specs=[blk(3), blk(16), blk(48), full((16, 16)), full((1, 16)),
                  full((384, 16)), full((24, 16)), full((16, 384)),
                  full((24, 384)), full((3, 6)), full((3, 6)), full((3, 1)),
                  full((3, 48)), full((6, 48)), full((1, 48)), full((48, 24))],
        out_specs=blk(16),
        out_shape=jax.ShapeDtypeStruct((E_PAD, 16), jnp.float32),
    )(ev_p, ef_p, od, fc2_w1, fc2_b1.reshape(1, 16), A2, B2,
      jnp.asarray(_RK2), jnp.asarray(_TU2),
      jnp.asarray(_SH['M1']), jnp.asarray(_SH['M2']), jnp.asarray(_SH['ONES3']),
      jnp.asarray(_SH['MD1']), jnp.asarray(_SH['MD2']),
      jnp.asarray(_SH['ONES16_48']), jnp.asarray(_SH['YP']))


def _tc_final(q0, q1, cntc, bn_gamma, bn_beta, lin2_w, lin2_b, skip):
    def body(q0_ref, q1_ref, c_ref, g_ref, b_ref, w_ref, wb_ref, s_ref, o_ref):
        o2 = (q0_ref[...] + q1_ref[...]) / c_ref[...]
        mean = jnp.mean(o2, axis=0, keepdims=True)
        var = jnp.mean((o2 - mean) ** 2, axis=0, keepdims=True)
        xn = (o2 - mean) / jnp.sqrt(var + 1e-5) * g_ref[...] + b_ref[...]
        h3 = _softplus(jnp.dot(_softplus(xn), w_ref[...],
                               preferred_element_type=jnp.float32) + wb_ref[...])
        o_ref[...] = h3 + s_ref[...]

    return pl.pallas_call(
        body,
        out_shape=jax.ShapeDtypeStruct((N_NODES, 128), jnp.float32),
    )(q0, q1, cntc, bn_gamma.reshape(1, 16), bn_beta.reshape(1, 16),
      lin2_w, lin2_b.reshape(1, 128), skip)


def kernel(edge_vec, node_feature, edge_index, edge_feature, lin1_w, lin1_b,
           fc1_w1, fc1_b1, fc1_w2, fc1_b2, fc2_w1, fc2_b1, fc2_w2, fc2_b2,
           bn_gamma, bn_beta, lin2_w, lin2_b):
    A1 = fc1_w2[:, _COL1].reshape(256, 24)
    B1 = fc1_b2[_COL1]
    A2 = fc2_w2[:, _COL2].reshape(384, 16)
    B2 = fc2_b2[_COL2]

    pe = E_PAD - N_EDGES
    ev_p = jnp.pad(edge_vec, ((0, pe), (0, 0)), constant_values=1.0)
    ef_p = jnp.pad(edge_feature, ((0, pe), (0, 0)))
    src2d = jnp.pad(edge_index[0], (0, pe)).reshape(E_PAD // GRP, GRP)
    dst2d = jnp.pad(edge_index[1], (0, pe)).reshape(E_PAD // GRP, GRP)

    h = _tc_h(node_feature, lin1_w, lin1_b)
    hd = _sc_gather(h, dst2d, 16)
    tp1e = _tc_tp1(ev_p, ef_p, hd, fc1_w1, fc1_b1, A1, B1)
    p = _sc_scatter_add(tp1e, src2d, 64)
    out1, cntc = _tc_out1(p[0], p[1], h)
    od = _sc_gather(out1, dst2d, 48)
    tp2 = _tc_tp2(ev_p, ef_p, od, fc2_w1, fc2_b1, A2, B2)
    q = _sc_scatter_add(tp2, src2d, 16)
    return _tc_final(q[0], q[1], cntc, bn_gamma, bn_beta,
                     lin2_w, lin2_b, node_feature)


# speedup vs baseline: 3.0890x; 1.0315x over previous
import functools

import jax
import jax.numpy as jnp
import numpy as np
from jax import lax
from jax.experimental import pallas as pl
from jax.experimental.pallas import tpu as pltpu
from jax.experimental.pallas import tpu_sc as plsc

N_NODES = 10000
N_EDGES = 160000
NW = 32
GRP = 128
GBLK = 8
CH = GRP * GBLK
E_PAD = 163840
EB = 2048
S3 = float(np.sqrt(3.0))
S5 = float(np.sqrt(5.0))
S15 = float(np.sqrt(15.0))
C1 = 1.0 / float(np.sqrt(16.0))
C2 = 1.0 / float(np.sqrt(24.0))


def _softplus(x):
    m = jnp.maximum(x, 0.0)
    return m + jnp.log(jnp.exp(x - m) + jnp.exp(-m))


def _np_consts():
    col1 = np.zeros((16, 24), dtype=np.int32)
    for u in range(16):
        for j in range(24):
            if j < 16:
                col1[u, j] = u * 16 + j
            elif j < 20:
                col1[u, j] = 256 + u * 4 + (j - 16)
            else:
                col1[u, j] = 320 + u * 4 + (j - 20)
    col2 = np.zeros((24, 16), dtype=np.int32)
    for u in range(24):
        for w in range(16):
            if u < 16:
                col2[u, w] = u * 16 + w
            elif u < 20:
                col2[u, w] = 256 + (u - 16) * 16 + w
            else:
                col2[u, w] = 320 + (u - 20) * 16 + w
    RK = np.repeat(np.eye(16, dtype=np.float32), 16, axis=1)
    TU = np.tile(np.eye(16, dtype=np.float32), (1, 16))
    RK2 = np.repeat(np.eye(16, dtype=np.float32), 24, axis=1)
    TU2 = np.tile(np.eye(24, dtype=np.float32), (1, 16))
    E1 = np.kron(np.eye(4, dtype=np.float32), np.ones((1, 3), np.float32))
    E2 = np.kron(np.eye(4, dtype=np.float32), np.ones((1, 5), np.float32))
    M1 = np.zeros((3, 6), np.float32)
    M2 = np.zeros((3, 6), np.float32)
    first = [0, 0, 1, 0, 2, 1]
    second = [2, 1, 1, 0, 2, 2]
    for c in range(6):
        M1[first[c], c] = 1.0
        M2[second[c], c] = 1.0
    C2m = np.zeros((6, 5), np.float32)
    C2m[0, 0] = S15
    C2m[1, 1] = S15
    C2m[2, 2] = S5
    C2m[3, 2] = -0.5 * S5
    C2m[4, 2] = -0.5 * S5
    C2m[5, 3] = S15
    C2m[4, 4] = 0.5 * S15
    C2m[3, 4] = -0.5 * S15
    TEXP = np.zeros((24, 64), np.float32)
    for j in range(16):
        TEXP[j, j] = 1.0
    for v in range(4):
        for m in range(3):
            TEXP[16 + v, 16 + 3 * v + m] = 1.0
        for c in range(5):
            TEXP[20 + v, 28 + 5 * v + c] = 1.0
    MQ1p = np.zeros((3, 64), np.float32)
    MQ2p = np.zeros((6, 64), np.float32)
    for v in range(4):
        for m in range(3):
            MQ1p[m, 16 + 3 * v + m] = S3
        for c in range(5):
            MQ2p[:, 28 + 5 * v + c] = C2m[:, c]
    ONES64 = np.zeros((1, 64), np.float32)
    ONES64[0, :16] = 1.0
    ONES64[0, 48] = 1.0
    ONE48 = np.zeros((1, 64), np.float32)
    ONE48[0, 48] = 1.0
    MD1 = np.zeros((3, 48), np.float32)
    MD2 = np.zeros((6, 48), np.float32)
    for v in range(4):
        for m in range(3):
            MD1[m, 16 + 3 * v + m] = S3
        for c in range(5):
            MD2[:, 28 + 5 * v + c] = C2m[:, c]
    ONES16_48 = np.zeros((1, 48), np.float32)
    ONES16_48[0, :16] = 1.0
    YP = np.zeros((48, 24), np.float32)
    for j in range(16):
        YP[j, j] = 1.0
    for v in range(4):
        for m in range(3):
            YP[16 + 3 * v + m, 16 + v] = 1.0 / S3
        for c in range(5):
            YP[28 + 5 * v + c, 20 + v] = 1.0 / S5
    sh = dict(M1=M1, M2=M2, TEXP=TEXP, MQ1p=MQ1p, MQ2p=MQ2p, ONES64=ONES64,
              ONE48=ONE48, MD1=MD1, MD2=MD2, ONES16_48=ONES16_48, YP=YP,
              ONES3=np.ones((3, 1), np.float32))
    return col1, col2, RK, TU, RK2, TU2, sh


_COL1, _COL2, _RK, _TU, _RK2, _TU2, _SH = _np_consts()


def _sc_gather(table, idx2d, d, gblk=GBLK):
    ch = GRP * gblk
    epw = E_PAD // NW
    gpw = epw // ch
    mesh = plsc.VectorSubcoreMesh(core_axis_name="c", subcore_axis_name="s")

    @functools.partial(
        pl.kernel, mesh=mesh,
        out_type=jax.ShapeDtypeStruct((E_PAD, d), jnp.float32),
        compiler_params=pltpu.CompilerParams(use_tc_tiling_on_sc=False),
        scratch_types=[
            pltpu.VMEM((2, gblk, GRP), jnp.int32),
            pltpu.VMEM((2, ch, d), jnp.float32),
            pltpu.SemaphoreType.DMA,
            pltpu.SemaphoreType.DMA,
            pltpu.SemaphoreType.DMA,
            pltpu.SemaphoreType.DMA,
            pltpu.SemaphoreType.DMA,
        ],
    )
    def k(table_hbm, idx_hbm, out_hbm, idx_v, rows_v,
          sem_i0, sem_i1, sem_g, sem_s0, sem_s1):
        wid = lax.axis_index("s") * 2 + lax.axis_index("c")
        base_g = wid * (epw // GRP)
        sem_i = (sem_i0, sem_i1)
        sem_s = (sem_s0, sem_s1)

        def idx_copy(i):
            return pltpu.make_async_copy(
                idx_hbm.at[pl.ds(base_g + i * gblk, gblk)],
                idx_v.at[i % 2], sem_i[i % 2])

        def out_copy(i):
            return pltpu.make_async_copy(
                rows_v.at[i % 2],
                out_hbm.at[pl.ds((base_g + i * gblk) * GRP, ch)],
                sem_s[i % 2])

        idx_copy(0).start()
        for i in range(gpw):
            b = i % 2
            if i + 1 < gpw:
                idx_copy(i + 1).start()
            idx_copy(i).wait()
            if i >= 2:
                out_copy(i - 2).wait()
            gathers = [
                pltpu.make_async_copy(
                    table_hbm.at[idx_v.at[b, j]],
                    rows_v.at[b, pl.ds(j * GRP, GRP)], sem_g)
                for j in range(gblk)
            ]
            for gcp in gathers:
                gcp.start()
            for gcp in gathers:
                gcp.wait()
            out_copy(i).start()
        for i in range(max(gpw - 2, 0), gpw):
            out_copy(i).wait()

    return k(table, idx2d)


def _sc_scatter_add(vals, idx2d, d, gblk=GBLK):
    ch = GRP * gblk
    epw = E_PAD // NW
    gpw = epw // ch
    rows_pt = N_NODES // 16
    mesh = plsc.VectorSubcoreMesh(core_axis_name="c", subcore_axis_name="s")
    zeros = jnp.zeros((N_NODES, d), jnp.float32)

    @functools.partial(
        pl.kernel, mesh=mesh,
        out_type=jax.ShapeDtypeStruct((2, N_NODES, d), jnp.float32),
        compiler_params=pltpu.CompilerParams(use_tc_tiling_on_sc=False),
        scratch_types=[
            pltpu.VMEM((3, gblk, GRP), jnp.int32),
            pltpu.VMEM((3, ch, d), jnp.float32),
            pltpu.VMEM_SHARED((N_NODES, d), jnp.float32),
            pltpu.SemaphoreType.DMA,
            pltpu.SemaphoreType.DMA,
            pltpu.SemaphoreType.DMA,
            pltpu.SemaphoreType.DMA,
            pltpu.SemaphoreType.DMA,
            pltpu.SemaphoreType.DMA,
            pltpu.SemaphoreType.DMA,
        ],
    )
    def k(vals_hbm, idx_hbm, zero_hbm, out_hbm, idx_v, rows_v, acc_sh,
          sem_i0, sem_i1, sem_i2, sem_v0, sem_v1, sem_v2, sem_sc):
        cid = lax.axis_index("c")
        sid = lax.axis_index("s")
        wid = sid * 2 + cid
        r0 = sid * rows_pt
        pltpu.sync_copy(zero_hbm.at[pl.ds(r0, rows_pt)],
                        acc_sh.at[pl.ds(r0, rows_pt)])
        plsc.subcore_barrier()
        base_g = wid * (epw // GRP)
        sem_i = (sem_i0, sem_i1, sem_i2)
        sem_v = (sem_v0, sem_v1, sem_v2)

        def idx_copy(i):
            return pltpu.make_async_copy(
                idx_hbm.at[pl.ds(base_g + i * gblk, gblk)],
                idx_v.at[i % 3], sem_i[i % 3])

        def val_copy(i):
            return pltpu.make_async_copy(
                vals_hbm.at[pl.ds((base_g + i * gblk) * GRP, ch)],
                rows_v.at[i % 3], sem_v[i % 3])

        def scatters(i):
            b = i % 3
            return [
                pltpu.make_async_copy(
                    rows_v.at[b, pl.ds(j * GRP, GRP)],
                    acc_sh.at[idx_v.at[b, j]], sem_sc)
                for j in range(gblk)
            ]

        idx_copy(0).start()
        val_copy(0).start()
        if gpw > 1:
            idx_copy(1).start()
            val_copy(1).start()
        for i in range(gpw):
            if i >= 2:
                for scp in scatters(i - 2):
                    scp.wait()
            if i >= 1 and i + 1 < gpw:
                idx_copy(i + 1).start()
                val_copy(i + 1).start()
            idx_copy(i).wait()
            val_copy(i).wait()
            for scp in scatters(i):
                scp.start(add=True)
        for i in range(max(gpw - 2, 0), gpw):
            for scp in scatters(i):
                scp.wait()
        plsc.subcore_barrier()
        pltpu.sync_copy(acc_sh.at[pl.ds(r0, rows_pt)],
                        out_hbm.at[cid, pl.ds(r0, rows_pt)])

    return k(vals, idx2d, zeros)


def _tc_h(node_feature, lin1_w, lin1_b):
    def body(nf, w, b, o):
        o[...] = jnp.dot(nf[...], w[...],
                         preferred_element_type=jnp.float32) + b[...]

    return pl.pallas_call(
        body,
        out_shape=jax.ShapeDtypeStruct((N_NODES, 16), jnp.float32),
    )(node_feature, lin1_w, lin1_b.reshape(1, 16))


def _dot(a, b):
    return jnp.dot(a, b, preferred_element_type=jnp.float32)


def _sh_expand(ev, mq1, mq2, ones_row, m1, m2, ones3):
    ss = _dot(ev * ev, ones3)
    rr = jnp.where(ss == 0.0, 1.0, ss)
    rinv = lax.rsqrt(rr)
    r2inv = 1.0 / rr
    evn = ev * rinv
    praw = (_dot(ev, m1) * _dot(ev, m2)) * r2inv
    return ones_row + _dot(evn, mq1) + _dot(praw, mq2)


def _tc_tp1(ev_p, ef_p, hd, fc1_w1, fc1_b1, A1, B1):
    grid = E_PAD // EB

    def body(ev_ref, ef_ref, hd_ref, w1_ref, b1_ref, a_ref, bb_ref,
             rk_ref, tu_ref, m1_ref, m2_ref, ones3_ref, mq1_ref, mq2_ref,
             ones64_ref, one48_ref, texp_ref, o_ref):
        i = pl.program_id(0)
        shE = _sh_expand(ev_ref[...], mq1_ref[...], mq2_ref[...],
                         ones64_ref[...], m1_ref[...], m2_ref[...],
                         ones3_ref[...])
        f1 = _softplus(_dot(ef_ref[...], w1_ref[...]) + b1_ref[...])
        hd_b = hd_ref[...]
        g = _dot(f1, rk_ref[...]) * _dot(hd_b, tu_ref[...])
        t = C1 * (_dot(g, a_ref[...]) + _dot(hd_b, bb_ref[...]))
        rows = i * EB + lax.broadcasted_iota(jnp.int32, (EB, 1), 0)
        mask = (rows < N_EDGES).astype(jnp.float32)
        o_ref[...] = (_dot(t, texp_ref[...]) + one48_ref[...]) * shE * mask

    full = lambda shape: pl.BlockSpec(shape, lambda i: (0, 0))
    blk = lambda d: pl.BlockSpec((EB, d), lambda i: (i, 0))
    return pl.pallas_call(
        body,
        grid=(grid,),
        in_specs=[blk(3), blk(16), blk(16), full((16, 16)), full((1, 16)),
                  full((256, 24)), full((16, 24)), full((16, 256)),
                  full((16, 256)), full((3, 6)), full((3, 6)), full((3, 1)),
                  full((3, 64)), full((6, 64)), full((1, 64)), full((1, 64)),
                  full((24, 64))],
        out_specs=blk(64),
        out_shape=jax.ShapeDtypeStruct((E_PAD, 64), jnp.float32),
    )(ev_p, ef_p, hd, fc1_w1, fc1_b1.reshape(1, 16), A1, B1,
      jnp.asarray(_RK), jnp.asarray(_TU),
      jnp.asarray(_SH['M1']), jnp.asarray(_SH['M2']), jnp.asarray(_SH['ONES3']),
      jnp.asarray(_SH['MQ1p']), jnp.asarray(_SH['MQ2p']),
      jnp.asarray(_SH['ONES64']), jnp.asarray(_SH['ONE48']),
      jnp.asarray(_SH['TEXP']))


def _tc_out1(p0, p1, h):
    def body(p0_ref, p1_ref, h_ref, o_ref, c_ref):
        s = p0_ref[...] + p1_ref[...]
        cnt = jnp.maximum(s[:, 48:49], 1.0)
        o = s[:, :48] / cnt
        o = o + jnp.concatenate(
            [h_ref[...], jnp.zeros((N_NODES, 32), jnp.float32)], axis=1)
        o_ref[...] = o
        c_ref[...] = cnt

    return pl.pallas_call(
        body,
        out_shape=(jax.ShapeDtypeStruct((N_NODES, 48), jnp.float32),
                   jax.ShapeDtypeStruct((N_NODES, 1), jnp.float32)),
    )(p0, p1, h)


def _tc_tp2(ev_p, ef_p, od, fc2_w1, fc2_b1, A2, B2):
    grid = E_PAD // EB

    def body(ev_ref, ef_ref, od_ref, w1_ref, b1_ref, a_ref, bb_ref,
             rk_ref, tu_ref, m1_ref, m2_ref, ones3_ref, md1_ref, md2_ref,
             ones48_ref, yp_ref, o_ref):
        i = pl.program_id(0)
        shD = _sh_expand(ev_ref[...], md1_ref[...], md2_ref[...],
                         ones48_ref[...], m1_ref[...], m2_ref[...],
                         ones3_ref[...])
        f2 = _softplus(_dot(ef_ref[...], w1_ref[...]) + b1_ref[...])
        yv = _dot(od_ref[...] * shD, yp_ref[...])
        g = _dot(f2, rk_ref[...]) * _dot(yv, tu_ref[...])
        t = C2 * (_dot(g, a_ref[...]) + _dot(yv, bb_ref[...]))
        rows = i * EB + lax.broadcasted_iota(jnp.int32, (EB, 1), 0)
        mask = (rows < N_EDGES).astype(jnp.float32)
        o_ref[...] = t * mask

    full = lambda shape: pl.BlockSpec(shape, lambda i: (0, 0))
    blk = lambda d: pl.BlockSpec((EB, d), lambda i: (i, 0))
    return pl.pallas_call(
        body,
        grid=(grid,),
        in_specs=[blk(3), blk(16), blk(48), full((16, 16)), full((1, 16)),
                  full((384, 16)), full((24, 16)), full((16, 384)),
                  full((24, 384)), full((3, 6)), full((3, 6)), full((3, 1)),
                  full((3, 48)), full((6, 48)), full((1, 48)), full((48, 24))],
        out_specs=blk(16),
        out_shape=jax.ShapeDtypeStruct((E_PAD, 16), jnp.float32),
    )(ev_p, ef_p, od, fc2_w1, fc2_b1.reshape(1, 16), A2, B2,
      jnp.asarray(_RK2), jnp.asarray(_TU2),
      jnp.asarray(_SH['M1']), jnp.asarray(_SH['M2']), jnp.asarray(_SH['ONES3']),
      jnp.asarray(_SH['MD1']), jnp.asarray(_SH['MD2']),
      jnp.asarray(_SH['ONES16_48']), jnp.asarray(_SH['YP']))


def _tc_final(q0, q1, cntc, bn_gamma, bn_beta, lin2_w, lin2_b, skip):
    def body(q0_ref, q1_ref, c_ref, g_ref, b_ref, w_ref, wb_ref, s_ref, o_ref):
        o2 = (q0_ref[...] + q1_ref[...]) / c_ref[...]
        mean = jnp.mean(o2, axis=0, keepdims=True)
        var = jnp.mean((o2 - mean) ** 2, axis=0, keepdims=True)
        xn = (o2 - mean) / jnp.sqrt(var + 1e-5) * g_ref[...] + b_ref[...]
        h3 = _softplus(jnp.dot(_softplus(xn), w_ref[...],
                               preferred_element_type=jnp.float32) + wb_ref[...])
        o_ref[...] = h3 + s_ref[...]

    return pl.pallas_call(
        body,
        out_shape=jax.ShapeDtypeStruct((N_NODES, 128), jnp.float32),
    )(q0, q1, cntc, bn_gamma.reshape(1, 16), bn_beta.reshape(1, 16),
      lin2_w, lin2_b.reshape(1, 128), skip)


def kernel(edge_vec, node_feature, edge_index, edge_feature, lin1_w, lin1_b,
           fc1_w1, fc1_b1, fc1_w2, fc1_b2, fc2_w1, fc2_b1, fc2_w2, fc2_b2,
           bn_gamma, bn_beta, lin2_w, lin2_b):
    A1 = fc1_w2[:, _COL1].reshape(256, 24)
    B1 = fc1_b2[_COL1]
    A2 = fc2_w2[:, _COL2].reshape(384, 16)
    B2 = fc2_b2[_COL2]

    pe = E_PAD - N_EDGES
    ev_p = jnp.pad(edge_vec, ((0, pe), (0, 0)), constant_values=1.0)
    ef_p = jnp.pad(edge_feature, ((0, pe), (0, 0)))
    src2d = jnp.pad(edge_index[0], (0, pe)).reshape(E_PAD // GRP, GRP)
    dst2d = jnp.pad(edge_index[1], (0, pe)).reshape(E_PAD // GRP, GRP)

    h = _tc_h(node_feature, lin1_w, lin1_b)
    hd = _sc_gather(h, dst2d, 16)
    tp1e = _tc_tp1(ev_p, ef_p, hd, fc1_w1, fc1_b1, A1, B1)
    p = _sc_scatter_add(tp1e, src2d, 64, gblk=2)
    out1, cntc = _tc_out1(p[0], p[1], h)
    od = _sc_gather(out1, dst2d, 48)
    tp2 = _tc_tp2(ev_p, ef_p, od, fc2_w1, fc2_b1, A2, B2)
    q = _sc_scatter_add(tp2, src2d, 16)
    return _tc_final(q[0], q[1], cntc, bn_gamma, bn_beta,
                     lin2_w, lin2_b, node_feature)


# speedup vs baseline: 3.1380x; 1.0159x over previous
import functools

import jax
import jax.numpy as jnp
import numpy as np
from jax import lax
from jax.experimental import pallas as pl
from jax.experimental.pallas import tpu as pltpu
from jax.experimental.pallas import tpu_sc as plsc

N_NODES = 10000
N_EDGES = 160000
NW = 32
GRP = 128
GBLK = 8
CH = GRP * GBLK
E_PAD = 163840
EB = 4096
S3 = float(np.sqrt(3.0))
S5 = float(np.sqrt(5.0))
S15 = float(np.sqrt(15.0))
C1 = 1.0 / float(np.sqrt(16.0))
C2 = 1.0 / float(np.sqrt(24.0))


def _softplus(x):
    m = jnp.maximum(x, 0.0)
    return m + jnp.log(jnp.exp(x - m) + jnp.exp(-m))


def _np_consts():
    col1 = np.zeros((16, 24), dtype=np.int32)
    for u in range(16):
        for j in range(24):
            if j < 16:
                col1[u, j] = u * 16 + j
            elif j < 20:
                col1[u, j] = 256 + u * 4 + (j - 16)
            else:
                col1[u, j] = 320 + u * 4 + (j - 20)
    col2 = np.zeros((24, 16), dtype=np.int32)
    for u in range(24):
        for w in range(16):
            if u < 16:
                col2[u, w] = u * 16 + w
            elif u < 20:
                col2[u, w] = 256 + (u - 16) * 16 + w
            else:
                col2[u, w] = 320 + (u - 20) * 16 + w
    RK = np.repeat(np.eye(16, dtype=np.float32), 16, axis=1)
    TU = np.tile(np.eye(16, dtype=np.float32), (1, 16))
    RK2 = np.repeat(np.eye(16, dtype=np.float32), 24, axis=1)
    TU2 = np.tile(np.eye(24, dtype=np.float32), (1, 16))
    E1 = np.kron(np.eye(4, dtype=np.float32), np.ones((1, 3), np.float32))
    E2 = np.kron(np.eye(4, dtype=np.float32), np.ones((1, 5), np.float32))
    M1 = np.zeros((3, 6), np.float32)
    M2 = np.zeros((3, 6), np.float32)
    first = [0, 0, 1, 0, 2, 1]
    second = [2, 1, 1, 0, 2, 2]
    for c in range(6):
        M1[first[c], c] = 1.0
        M2[second[c], c] = 1.0
    C2m = np.zeros((6, 5), np.float32)
    C2m[0, 0] = S15
    C2m[1, 1] = S15
    C2m[2, 2] = S5
    C2m[3, 2] = -0.5 * S5
    C2m[4, 2] = -0.5 * S5
    C2m[5, 3] = S15
    C2m[4, 4] = 0.5 * S15
    C2m[3, 4] = -0.5 * S15
    TEXP = np.zeros((24, 64), np.float32)
    for j in range(16):
        TEXP[j, j] = 1.0
    for v in range(4):
        for m in range(3):
            TEXP[16 + v, 16 + 3 * v + m] = 1.0
        for c in range(5):
            TEXP[20 + v, 28 + 5 * v + c] = 1.0
    MQ1p = np.zeros((3, 64), np.float32)
    MQ2p = np.zeros((6, 64), np.float32)
    for v in range(4):
        for m in range(3):
            MQ1p[m, 16 + 3 * v + m] = S3
        for c in range(5):
            MQ2p[:, 28 + 5 * v + c] = C2m[:, c]
    ONES64 = np.zeros((1, 64), np.float32)
    ONES64[0, :16] = 1.0
    ONES64[0, 48] = 1.0
    ONE48 = np.zeros((1, 64), np.float32)
    ONE48[0, 48] = 1.0
    MD1 = np.zeros((3, 48), np.float32)
    MD2 = np.zeros((6, 48), np.float32)
    for v in range(4):
        for m in range(3):
            MD1[m, 16 + 3 * v + m] = S3
        for c in range(5):
            MD2[:, 28 + 5 * v + c] = C2m[:, c]
    ONES16_48 = np.zeros((1, 48), np.float32)
    ONES16_48[0, :16] = 1.0
    YP = np.zeros((48, 24), np.float32)
    for j in range(16):
        YP[j, j] = 1.0
    for v in range(4):
        for m in range(3):
            YP[16 + 3 * v + m, 16 + v] = 1.0 / S3
        for c in range(5):
            YP[28 + 5 * v + c, 20 + v] = 1.0 / S5
    sh = dict(M1=M1, M2=M2, TEXP=TEXP, MQ1p=MQ1p, MQ2p=MQ2p, ONES64=ONES64,
              ONE48=ONE48, MD1=MD1, MD2=MD2, ONES16_48=ONES16_48, YP=YP,
              ONES3=np.ones((3, 1), np.float32))
    return col1, col2, RK, TU, RK2, TU2, sh


_COL1, _COL2, _RK, _TU, _RK2, _TU2, _SH = _np_consts()


def _sc_gather(table, idx2d, d, gblk=GBLK):
    ch = GRP * gblk
    epw = E_PAD // NW
    gpw = epw // ch
    mesh = plsc.VectorSubcoreMesh(core_axis_name="c", subcore_axis_name="s")

    @functools.partial(
        pl.kernel, mesh=mesh,
        out_type=jax.ShapeDtypeStruct((E_PAD, d), jnp.float32),
        compiler_params=pltpu.CompilerParams(use_tc_tiling_on_sc=False),
        scratch_types=[
            pltpu.VMEM((2, gblk, GRP), jnp.int32),
            pltpu.VMEM((2, ch, d), jnp.float32),
            pltpu.SemaphoreType.DMA,
            pltpu.SemaphoreType.DMA,
            pltpu.SemaphoreType.DMA,
            pltpu.SemaphoreType.DMA,
            pltpu.SemaphoreType.DMA,
        ],
    )
    def k(table_hbm, idx_hbm, out_hbm, idx_v, rows_v,
          sem_i0, sem_i1, sem_g, sem_s0, sem_s1):
        wid = lax.axis_index("s") * 2 + lax.axis_index("c")
        base_g = wid * (epw // GRP)
        sem_i = (sem_i0, sem_i1)
        sem_s = (sem_s0, sem_s1)

        def idx_copy(i):
            return pltpu.make_async_copy(
                idx_hbm.at[pl.ds(base_g + i * gblk, gblk)],
                idx_v.at[i % 2], sem_i[i % 2])

        def out_copy(i):
            return pltpu.make_async_copy(
                rows_v.at[i % 2],
                out_hbm.at[pl.ds((base_g + i * gblk) * GRP, ch)],
                sem_s[i % 2])

        idx_copy(0).start()
        for i in range(gpw):
            b = i % 2
            if i + 1 < gpw:
                idx_copy(i + 1).start()
            idx_copy(i).wait()
            if i >= 2:
                out_copy(i - 2).wait()
            gathers = [
                pltpu.make_async_copy(
                    table_hbm.at[idx_v.at[b, j]],
                    rows_v.at[b, pl.ds(j * GRP, GRP)], sem_g)
                for j in range(gblk)
            ]
            for gcp in gathers:
                gcp.start()
            for gcp in gathers:
                gcp.wait()
            out_copy(i).start()
        for i in range(max(gpw - 2, 0), gpw):
            out_copy(i).wait()

    return k(table, idx2d)


def _sc_scatter_add(vals, idx2d, d, gblk=GBLK):
    ch = GRP * gblk
    epw = E_PAD // NW
    gpw = epw // ch
    rows_pt = N_NODES // 16
    mesh = plsc.VectorSubcoreMesh(core_axis_name="c", subcore_axis_name="s")
    zeros = jnp.zeros((N_NODES, d), jnp.float32)

    @functools.partial(
        pl.kernel, mesh=mesh,
        out_type=jax.ShapeDtypeStruct((2, N_NODES, d), jnp.float32),
        compiler_params=pltpu.CompilerParams(use_tc_tiling_on_sc=False),
        scratch_types=[
            pltpu.VMEM((3, gblk, GRP), jnp.int32),
            pltpu.VMEM((3, ch, d), jnp.float32),
            pltpu.VMEM_SHARED((N_NODES, d), jnp.float32),
            pltpu.SemaphoreType.DMA,
            pltpu.SemaphoreType.DMA,
            pltpu.SemaphoreType.DMA,
            pltpu.SemaphoreType.DMA,
            pltpu.SemaphoreType.DMA,
            pltpu.SemaphoreType.DMA,
            pltpu.SemaphoreType.DMA,
        ],
    )
    def k(vals_hbm, idx_hbm, zero_hbm, out_hbm, idx_v, rows_v, acc_sh,
          sem_i0, sem_i1, sem_i2, sem_v0, sem_v1, sem_v2, sem_sc):
        cid = lax.axis_index("c")
        sid = lax.axis_index("s")
        wid = sid * 2 + cid
        r0 = sid * rows_pt
        pltpu.sync_copy(zero_hbm.at[pl.ds(r0, rows_pt)],
                        acc_sh.at[pl.ds(r0, rows_pt)])
        plsc.subcore_barrier()
        base_g = wid * (epw // GRP)
        sem_i = (sem_i0, sem_i1, sem_i2)
        sem_v = (sem_v0, sem_v1, sem_v2)

        def idx_copy(i):
            return pltpu.make_async_copy(
                idx_hbm.at[pl.ds(base_g + i * gblk, gblk)],
                idx_v.at[i % 3], sem_i[i % 3])

        def val_copy(i):
            return pltpu.make_async_copy(
                vals_hbm.at[pl.ds((base_g + i * gblk) * GRP, ch)],
                rows_v.at[i % 3], sem_v[i % 3])

        def scatters(i):
            b = i % 3
            return [
                pltpu.make_async_copy(
                    rows_v.at[b, pl.ds(j * GRP, GRP)],
                    acc_sh.at[idx_v.at[b, j]], sem_sc)
                for j in range(gblk)
            ]

        idx_copy(0).start()
        val_copy(0).start()
        if gpw > 1:
            idx_copy(1).start()
            val_copy(1).start()
        for i in range(gpw):
            if i >= 2:
                for scp in scatters(i - 2):
                    scp.wait()
            if i >= 1 and i + 1 < gpw:
                idx_copy(i + 1).start()
                val_copy(i + 1).start()
            idx_copy(i).wait()
            val_copy(i).wait()
            for scp in scatters(i):
                scp.start(add=True)
        for i in range(max(gpw - 2, 0), gpw):
            for scp in scatters(i):
                scp.wait()
        plsc.subcore_barrier()
        pltpu.sync_copy(acc_sh.at[pl.ds(r0, rows_pt)],
                        out_hbm.at[cid, pl.ds(r0, rows_pt)])

    return k(vals, idx2d, zeros)


def _tc_h(node_feature, lin1_w, lin1_b):
    def body(nf, w, b, o):
        o[...] = jnp.dot(nf[...], w[...],
                         preferred_element_type=jnp.float32) + b[...]

    return pl.pallas_call(
        body,
        out_shape=jax.ShapeDtypeStruct((N_NODES, 16), jnp.float32),
    )(node_feature, lin1_w, lin1_b.reshape(1, 16))


def _dot(a, b):
    return jnp.dot(a, b, preferred_element_type=jnp.float32)


def _sh_expand(ev, mq1, mq2, ones_row, m1, m2, ones3):
    ss = _dot(ev * ev, ones3)
    rr = jnp.where(ss == 0.0, 1.0, ss)
    rinv = lax.rsqrt(rr)
    r2inv = 1.0 / rr
    evn = ev * rinv
    praw = (_dot(ev, m1) * _dot(ev, m2)) * r2inv
    return ones_row + _dot(evn, mq1) + _dot(praw, mq2)


def _tc_tp1(ev_p, ef_p, hd, fc1_w1, fc1_b1, A1, B1):
    grid = E_PAD // EB

    def body(ev_ref, ef_ref, hd_ref, w1_ref, b1_ref, a_ref, bb_ref,
             rk_ref, tu_ref, m1_ref, m2_ref, ones3_ref, mq1_ref, mq2_ref,
             ones64_ref, one48_ref, texp_ref, o_ref):
        i = pl.program_id(0)
        shE = _sh_expand(ev_ref[...], mq1_ref[...], mq2_ref[...],
                         ones64_ref[...], m1_ref[...], m2_ref[...],
                         ones3_ref[...])
        f1 = _softplus(_dot(ef_ref[...], w1_ref[...]) + b1_ref[...])
        hd_b = hd_ref[...]
        g = _dot(f1, rk_ref[...]) * _dot(hd_b, tu_ref[...])
        t = C1 * (_dot(g, a_ref[...]) + _dot(hd_b, bb_ref[...]))
        rows = i * EB + lax.broadcasted_iota(jnp.int32, (EB, 1), 0)
        mask = (rows < N_EDGES).astype(jnp.float32)
        o_ref[...] = (_dot(t, texp_ref[...]) + one48_ref[...]) * shE * mask

    full = lambda shape: pl.BlockSpec(shape, lambda i: (0, 0))
    blk = lambda d: pl.BlockSpec((EB, d), lambda i: (i, 0))
    return pl.pallas_call(
        body,
        grid=(grid,),
        in_specs=[blk(3), blk(16), blk(16), full((16, 16)), full((1, 16)),
                  full((256, 24)), full((16, 24)), full((16, 256)),
                  full((16, 256)), full((3, 6)), full((3, 6)), full((3, 1)),
                  full((3, 64)), full((6, 64)), full((1, 64)), full((1, 64)),
                  full((24, 64))],
        out_specs=blk(64),
        out_shape=jax.ShapeDtypeStruct((E_PAD, 64), jnp.float32),
    )(ev_p, ef_p, hd, fc1_w1, fc1_b1.reshape(1, 16), A1, B1,
      jnp.asarray(_RK), jnp.asarray(_TU),
      jnp.asarray(_SH['M1']), jnp.asarray(_SH['M2']), jnp.asarray(_SH['ONES3']),
      jnp.asarray(_SH['MQ1p']), jnp.asarray(_SH['MQ2p']),
      jnp.asarray(_SH['ONES64']), jnp.asarray(_SH['ONE48']),
      jnp.asarray(_SH['TEXP']))


def _tc_out1(p0, p1, h):
    def body(p0_ref, p1_ref, h_ref, o_ref, c_ref):
        s = p0_ref[...] + p1_ref[...]
        cnt = jnp.maximum(s[:, 48:49], 1.0)
        o = s[:, :48] / cnt
        o = o + jnp.concatenate(
            [h_ref[...], jnp.zeros((N_NODES, 32), jnp.float32)], axis=1)
        o_ref[...] = o
        c_ref[...] = cnt

    return pl.pallas_call(
        body,
        out_shape=(jax.ShapeDtypeStruct((N_NODES, 48), jnp.float32),
                   jax.ShapeDtypeStruct((N_NODES, 1), jnp.float32)),
    )(p0, p1, h)


def _tc_tp2(ev_p, ef_p, od, fc2_w1, fc2_b1, A2, B2):
    grid = E_PAD // EB

    def body(ev_ref, ef_ref, od_ref, w1_ref, b1_ref, a_ref, bb_ref,
             rk_ref, tu_ref, m1_ref, m2_ref, ones3_ref, md1_ref, md2_ref,
             ones48_ref, yp_ref, o_ref):
        i = pl.program_id(0)
        shD = _sh_expand(ev_ref[...], md1_ref[...], md2_ref[...],
                         ones48_ref[...], m1_ref[...], m2_ref[...],
                         ones3_ref[...])
        f2 = _softplus(_dot(ef_ref[...], w1_ref[...]) + b1_ref[...])
        yv = _dot(od_ref[...] * shD, yp_ref[...])
        g = _dot(f2, rk_ref[...]) * _dot(yv, tu_ref[...])
        t = C2 * (_dot(g, a_ref[...]) + _dot(yv, bb_ref[...]))
        rows = i * EB + lax.broadcasted_iota(jnp.int32, (EB, 1), 0)
        mask = (rows < N_EDGES).astype(jnp.float32)
        o_ref[...] = t * mask

    full = lambda shape: pl.BlockSpec(shape, lambda i: (0, 0))
    blk = lambda d: pl.BlockSpec((EB, d), lambda i: (i, 0))
    return pl.pallas_call(
        body,
        grid=(grid,),
        in_specs=[blk(3), blk(16), blk(48), full((16, 16)), full((1, 16)),
                  full((384, 16)), full((24, 16)), full((16, 384)),
                  full((24, 384)), full((3, 6)), full((3, 6)), full((3, 1)),
                  full((3, 48)), full((6, 48)), full((1, 48)), full((48, 24))],
        out_specs=blk(16),
        out_shape=jax.ShapeDtypeStruct((E_PAD, 16), jnp.float32),
    )(ev_p, ef_p, od, fc2_w1, fc2_b1.reshape(1, 16), A2, B2,
      jnp.asarray(_RK2), jnp.asarray(_TU2),
      jnp.asarray(_SH['M1']), jnp.asarray(_SH['M2']), jnp.asarray(_SH['ONES3']),
      jnp.asarray(_SH['MD1']), jnp.asarray(_SH['MD2']),
      jnp.asarray(_SH['ONES16_48']), jnp.asarray(_SH['YP']))


def _tc_final(q0, q1, cntc, bn_gamma, bn_beta, lin2_w, lin2_b, skip):
    def body(q0_ref, q1_ref, c_ref, g_ref, b_ref, w_ref, wb_ref, s_ref, o_ref):
        o2 = (q0_ref[...] + q1_ref[...]) / c_ref[...]
        mean = jnp.mean(o2, axis=0, keepdims=True)
        var = jnp.mean((o2 - mean) ** 2, axis=0, keepdims=True)
        xn = (o2 - mean) / jnp.sqrt(var + 1e-5) * g_ref[...] + b_ref[...]
        h3 = _softplus(jnp.dot(_softplus(xn), w_ref[...],
                               preferred_element_type=jnp.float32) + wb_ref[...])
        o_ref[...] = h3 + s_ref[...]

    return pl.pallas_call(
        body,
        out_shape=jax.ShapeDtypeStruct((N_NODES, 128), jnp.float32),
    )(q0, q1, cntc, bn_gamma.reshape(1, 16), bn_beta.reshape(1, 16),
      lin2_w, lin2_b.reshape(1, 128), skip)


def kernel(edge_vec, node_feature, edge_index, edge_feature, lin1_w, lin1_b,
           fc1_w1, fc1_b1, fc1_w2, fc1_b2, fc2_w1, fc2_b1, fc2_w2, fc2_b2,
           bn_gamma, bn_beta, lin2_w, lin2_b):
    A1 = fc1_w2[:, _COL1].reshape(256, 24)
    B1 = fc1_b2[_COL1]
    A2 = fc2_w2[:, _COL2].reshape(384, 16)
    B2 = fc2_b2[_COL2]

    pe = E_PAD - N_EDGES
    ev_p = jnp.pad(edge_vec, ((0, pe), (0, 0)), constant_values=1.0)
    ef_p = jnp.pad(edge_feature, ((0, pe), (0, 0)))
    src2d = jnp.pad(edge_index[0], (0, pe)).reshape(E_PAD // GRP, GRP)
    dst2d = jnp.pad(edge_index[1], (0, pe)).reshape(E_PAD // GRP, GRP)

    h = _tc_h(node_feature, lin1_w, lin1_b)
    hd = _sc_gather(h, dst2d, 16)
    tp1e = _tc_tp1(ev_p, ef_p, hd, fc1_w1, fc1_b1, A1, B1)
    p = _sc_scatter_add(tp1e, src2d, 64, gblk=2)
    out1, cntc = _tc_out1(p[0], p[1], h)
    od = _sc_gather(out1, dst2d, 48)
    tp2 = _tc_tp2(ev_p, ef_p, od, fc2_w1, fc2_b1, A2, B2)
    q = _sc_scatter_add(tp2, src2d, 16)
    return _tc_final(q[0], q[1], cntc, bn_gamma, bn_beta,
                     lin2_w, lin2_b, node_feature)


# speedup vs baseline: 5.2197x; 1.6634x over previous
import functools

import jax
import jax.numpy as jnp
import numpy as np
from jax import lax
from jax.experimental import pallas as pl
from jax.experimental.pallas import tpu as pltpu
from jax.experimental.pallas import tpu_sc as plsc

N_NODES = 10000
N_EDGES = 160000
NW = 32
GRP = 128
IDX_PAD = 163840
EB = 4000
S3 = float(np.sqrt(3.0))
S5 = float(np.sqrt(5.0))
S15 = float(np.sqrt(15.0))
C1 = 1.0 / float(np.sqrt(16.0))
C2 = 1.0 / float(np.sqrt(24.0))


def _softplus(x):
    m = jnp.maximum(x, 0.0)
    return m + jnp.log(jnp.exp(x - m) + jnp.exp(-m))


def _dot(a, b):
    return jnp.dot(a, b, preferred_element_type=jnp.float32)


def _np_consts():
    col1 = np.zeros((16, 24), dtype=np.int32)
    for u in range(16):
        for j in range(24):
            if j < 16:
                col1[u, j] = u * 16 + j
            elif j < 20:
                col1[u, j] = 256 + u * 4 + (j - 16)
            else:
                col1[u, j] = 320 + u * 4 + (j - 20)
    col2 = np.zeros((24, 16), dtype=np.int32)
    for u in range(24):
        for w in range(16):
            if u < 16:
                col2[u, w] = u * 16 + w
            elif u < 20:
                col2[u, w] = 256 + (u - 16) * 16 + w
            else:
                col2[u, w] = 320 + (u - 20) * 16 + w
    RK = np.repeat(np.eye(16, dtype=np.float32), 16, axis=1)
    TU = np.tile(np.eye(16, dtype=np.float32), (1, 16))
    RK2 = np.repeat(np.eye(16, dtype=np.float32), 24, axis=1)
    TU2 = np.tile(np.eye(24, dtype=np.float32), (1, 16))
    M1 = np.zeros((3, 6), np.float32)
    M2 = np.zeros((3, 6), np.float32)
    first = [0, 0, 1, 0, 2, 1]
    second = [2, 1, 1, 0, 2, 2]
    for c in range(6):
        M1[first[c], c] = 1.0
        M2[second[c], c] = 1.0
    C2m = np.zeros((6, 5), np.float32)
    C2m[0, 0] = S15
    C2m[1, 1] = S15
    C2m[2, 2] = S5
    C2m[3, 2] = -0.5 * S5
    C2m[4, 2] = -0.5 * S5
    C2m[5, 3] = S15
    C2m[4, 4] = 0.5 * S15
    C2m[3, 4] = -0.5 * S15
    TEXP = np.zeros((24, 128), np.float32)
    for j in range(16):
        TEXP[j, j] = 1.0
    for v in range(4):
        for m in range(3):
            TEXP[16 + v, 16 + 3 * v + m] = 1.0
        for c in range(5):
            TEXP[20 + v, 28 + 5 * v + c] = 1.0
    MQ1p = np.zeros((3, 128), np.float32)
    MQ2p = np.zeros((6, 128), np.float32)
    for v in range(4):
        for m in range(3):
            MQ1p[m, 16 + 3 * v + m] = S3
        for c in range(5):
            MQ2p[:, 28 + 5 * v + c] = C2m[:, c]
    ONES64 = np.zeros((1, 128), np.float32)
    ONES64[0, :16] = 1.0
    ONES64[0, 48] = 1.0
    ONE48 = np.zeros((1, 128), np.float32)
    ONE48[0, 48] = 1.0
    MD1 = np.zeros((3, 128), np.float32)
    MD2 = np.zeros((6, 128), np.float32)
    for v in range(4):
        for m in range(3):
            MD1[m, 16 + 3 * v + m] = S3
        for c in range(5):
            MD2[:, 28 + 5 * v + c] = C2m[:, c]
    ONES16_48 = np.zeros((1, 128), np.float32)
    ONES16_48[0, :16] = 1.0
    YP = np.zeros((128, 24), np.float32)
    for j in range(16):
        YP[j, j] = 1.0
    for v in range(4):
        for m in range(3):
            YP[16 + 3 * v + m, 16 + v] = 1.0 / S3
        for c in range(5):
            YP[28 + 5 * v + c, 20 + v] = 1.0 / S5
    TU128 = np.zeros((128, 256), np.float32)
    TU128[:16] = TU
    MASK16 = np.zeros((1, 128), np.float32)
    MASK16[0, :16] = 1.0
    MASK48 = np.zeros((1, 128), np.float32)
    MASK48[0, :48] = 1.0
    sh = dict(M1=M1, M2=M2, TEXP=TEXP, MQ1p=MQ1p, MQ2p=MQ2p, ONES64=ONES64,
              ONE48=ONE48, MD1=MD1, MD2=MD2, ONES16_48=ONES16_48, YP=YP,
              TU128=TU128, MASK16=MASK16, MASK48=MASK48,
              ONES3=np.ones((3, 1), np.float32))
    return col1, col2, RK, TU, RK2, TU2, sh


_COL1, _COL2, _RK, _TU, _RK2, _TU2, _SH = _np_consts()


def _sc_gather(table, idx2d, d, gblk):
    ch = GRP * gblk
    cpw = (IDX_PAD // ch) // NW
    real = N_EDGES // ch
    assert real * ch == N_EDGES
    mesh = plsc.VectorSubcoreMesh(core_axis_name="c", subcore_axis_name="s")

    @functools.partial(
        pl.kernel, mesh=mesh,
        out_type=jax.ShapeDtypeStruct((N_EDGES, 128), jnp.float32),
        compiler_params=pltpu.CompilerParams(use_tc_tiling_on_sc=False),
        scratch_types=[
            pltpu.VMEM((2, gblk, GRP), jnp.int32),
            pltpu.VMEM((2, ch, d), jnp.float32),
            pltpu.SemaphoreType.DMA,
            pltpu.SemaphoreType.DMA,
            pltpu.SemaphoreType.DMA,
            pltpu.SemaphoreType.DMA,
            pltpu.SemaphoreType.DMA,
        ],
    )
    def k(table_hbm, idx_hbm, out_hbm, idx_v, rows_v,
          sem_i0, sem_i1, sem_g, sem_s0, sem_s1):
        wid = lax.axis_index("s") * 2 + lax.axis_index("c")
        sem_i = (sem_i0, sem_i1)
        sem_s = (sem_s0, sem_s1)

        def pred(c):
            return wid * cpw + c < real

        def idx_copy(c):
            g = (wid * cpw + c) * gblk
            return pltpu.make_async_copy(
                idx_hbm.at[pl.ds(g, gblk)], idx_v.at[c % 2], sem_i[c % 2])

        def out_copy(c):
            eoff = (wid * cpw + c) * ch
            return pltpu.make_async_copy(
                rows_v.at[c % 2],
                out_hbm.at[pl.ds(eoff, ch), pl.ds(0, d)], sem_s[c % 2])

        @pl.when(pred(0))
        def _():
            idx_copy(0).start()

        for c in range(cpw):
            b = c % 2
            if c + 1 < cpw:
                @pl.when(pred(c + 1))
                def _(c=c):
                    idx_copy(c + 1).start()

            @pl.when(pred(c))
            def _(c=c, b=b):
                idx_copy(c).wait()
                if c >= 2:
                    out_copy(c - 2).wait()
                gathers = [
                    pltpu.make_async_copy(
                        table_hbm.at[idx_v.at[b, j]],
                        rows_v.at[b, pl.ds(j * GRP, GRP)], sem_g)
                    for j in range(gblk)
                ]
                for gcp in gathers:
                    gcp.start()
                for gcp in gathers:
                    gcp.wait()
                out_copy(c).start()

        for c in range(max(cpw - 2, 0), cpw):
            @pl.when(pred(c))
            def _(c=c):
                out_copy(c).wait()

    return k(table, idx2d)


def _sc_scatter_add(vals, idx2d, d, gblk):
    ch = GRP * gblk
    cpw = (IDX_PAD // ch) // NW
    real = N_EDGES // ch
    assert real * ch == N_EDGES
    rows_pt = N_NODES // 16
    mesh = plsc.VectorSubcoreMesh(core_axis_name="c", subcore_axis_name="s")
    zeros = jnp.zeros((N_NODES, d), jnp.float32)

    @functools.partial(
        pl.kernel, mesh=mesh,
        out_type=jax.ShapeDtypeStruct((2, N_NODES, 128), jnp.float32),
        compiler_params=pltpu.CompilerParams(use_tc_tiling_on_sc=False),
        scratch_types=[
            pltpu.VMEM((3, gblk, GRP), jnp.int32),
            pltpu.VMEM((3, ch, d), jnp.float32),
            pltpu.VMEM_SHARED((N_NODES, d), jnp.float32),
            pltpu.SemaphoreType.DMA,
            pltpu.SemaphoreType.DMA,
            pltpu.SemaphoreType.DMA,
            pltpu.SemaphoreType.DMA,
            pltpu.SemaphoreType.DMA,
            pltpu.SemaphoreType.DMA,
            pltpu.SemaphoreType.DMA,
        ],
    )
    def k(vals_hbm, idx_hbm, zero_hbm, out_hbm, idx_v, rows_v, acc_sh,
          sem_i0, sem_i1, sem_i2, sem_v0, sem_v1, sem_v2, sem_sc):
        cid = lax.axis_index("c")
        sid = lax.axis_index("s")
        wid = sid * 2 + cid
        r0 = sid * rows_pt
        pltpu.sync_copy(zero_hbm.at[pl.ds(r0, rows_pt)],
                        acc_sh.at[pl.ds(r0, rows_pt)])
        plsc.subcore_barrier()
        sem_i = (sem_i0, sem_i1, sem_i2)
        sem_v = (sem_v0, sem_v1, sem_v2)

        def pred(c):
            return wid * cpw + c < real

        def idx_copy(c):
            g = (wid * cpw + c) * gblk
            return pltpu.make_async_copy(
                idx_hbm.at[pl.ds(g, gblk)], idx_v.at[c % 3], sem_i[c % 3])

        def val_copy(c):
            eoff = (wid * cpw + c) * ch
            return pltpu.make_async_copy(
                vals_hbm.at[pl.ds(eoff, ch), pl.ds(0, d)],
                rows_v.at[c % 3], sem_v[c % 3])

        def scatters(c):
            b = c % 3
            return [
                pltpu.make_async_copy(
                    rows_v.at[b, pl.ds(j * GRP, GRP)],
                    acc_sh.at[idx_v.at[b, j]], sem_sc)
                for j in range(gblk)
            ]

        @pl.when(pred(0))
        def _():
            idx_copy(0).start()
            val_copy(0).start()

        if cpw > 1:
            @pl.when(pred(1))
            def _():
                idx_copy(1).start()
                val_copy(1).start()

        for c in range(cpw):
            if c >= 2:
                @pl.when(pred(c - 2))
                def _(c=c):
                    for scp in scatters(c - 2):
                        scp.wait()
            if c >= 1 and c + 1 < cpw:
                @pl.when(pred(c + 1))
                def _(c=c):
                    idx_copy(c + 1).start()
                    val_copy(c + 1).start()

            @pl.when(pred(c))
            def _(c=c):
                idx_copy(c).wait()
                val_copy(c).wait()
                for scp in scatters(c):
                    scp.start(add=True)

        for c in range(max(cpw - 2, 0), cpw):
            @pl.when(pred(c))
            def _(c=c):
                for scp in scatters(c):
                    scp.wait()
        plsc.subcore_barrier()
        pltpu.sync_copy(acc_sh.at[pl.ds(r0, rows_pt)],
                        out_hbm.at[cid, pl.ds(r0, rows_pt), pl.ds(0, d)])

    return k(vals, idx2d, zeros)


def _tc_h(node_feature, lin1_w, lin1_b):
    def body(nf, w, b, o):
        o[...] = _dot(nf[...], w[...]) + b[...]

    return pl.pallas_call(
        body,
        out_shape=jax.ShapeDtypeStruct((N_NODES, 16), jnp.float32),
    )(node_feature, lin1_w, lin1_b.reshape(1, 16))


def _sh_expand(ev, mq1, mq2, ones_row, m1, m2, ones3):
    ss = _dot(ev * ev, ones3)
    rr = jnp.where(ss == 0.0, 1.0, ss)
    rinv = lax.rsqrt(rr)
    r2inv = 1.0 / rr
    evn = ev * rinv
    praw = (_dot(ev, m1) * _dot(ev, m2)) * r2inv
    return ones_row + _dot(evn, mq1) + _dot(praw, mq2)


def _tc_tp1(ev, ef, hd, fc1_w1, fc1_b1, A1, B1):
    grid = N_EDGES // EB

    def body(ev_ref, ef_ref, hd_ref, w1_ref, b1_ref, a_ref, bb_ref,
             rk_ref, tu_ref, m1_ref, m2_ref, ones3_ref, mq1_ref, mq2_ref,
             ones64_ref, one48_ref, texp_ref, msk_ref, o_ref):
        shE = _sh_expand(ev_ref[...], mq1_ref[...], mq2_ref[...],
                         ones64_ref[...], m1_ref[...], m2_ref[...],
                         ones3_ref[...])
        f1 = _softplus(_dot(ef_ref[...], w1_ref[...]) + b1_ref[...])
        hd_b = jnp.where(msk_ref[...] > 0.0, hd_ref[...], 0.0)
        g = _dot(f1, rk_ref[...]) * _dot(hd_b, tu_ref[...])
        t = C1 * (_dot(g, a_ref[...]) + _dot(hd_b, bb_ref[...]))
        o_ref[...] = (_dot(t, texp_ref[...]) + one48_ref[...]) * shE

    full = lambda shape: pl.BlockSpec(shape, lambda i: (0, 0))
    blk = lambda d: pl.BlockSpec((EB, d), lambda i: (i, 0))
    return pl.pallas_call(
        body,
        grid=(grid,),
        in_specs=[blk(3), blk(16), blk(128), full((16, 16)), full((1, 16)),
                  full((256, 24)), full((128, 24)), full((16, 256)),
                  full((128, 256)), full((3, 6)), full((3, 6)), full((3, 1)),
                  full((3, 128)), full((6, 128)), full((1, 128)),
                  full((1, 128)), full((24, 128)), full((1, 128))],
        out_specs=blk(128),
        out_shape=jax.ShapeDtypeStruct((N_EDGES, 128), jnp.float32),
    )(ev, ef, hd, fc1_w1, fc1_b1.reshape(1, 16), A1,
      jnp.pad(B1, ((0, 112), (0, 0))),
      jnp.asarray(_RK), jnp.asarray(_SH['TU128']),
      jnp.asarray(_SH['M1']), jnp.asarray(_SH['M2']), jnp.asarray(_SH['ONES3']),
      jnp.asarray(_SH['MQ1p']), jnp.asarray(_SH['MQ2p']),
      jnp.asarray(_SH['ONES64']), jnp.asarray(_SH['ONE48']),
      jnp.asarray(_SH['TEXP']), jnp.asarray(_SH['MASK16']))


def _tc_out1(p, h):
    def body(p_ref, h_ref, o_ref, c_ref):
        s = p_ref[0] + p_ref[1]
        cnt = jnp.maximum(s[:, 48:49], 1.0)
        o = s[:, :48] / cnt
        o = o + jnp.concatenate(
            [h_ref[...], jnp.zeros((N_NODES, 32), jnp.float32)], axis=1)
        o_ref[...] = o
        c_ref[...] = cnt

    return pl.pallas_call(
        body,
        in_specs=[pl.BlockSpec((2, N_NODES, 128), lambda: (0, 0, 0)),
                  pl.BlockSpec((N_NODES, 16), lambda: (0, 0))],
        out_shape=(jax.ShapeDtypeStruct((N_NODES, 48), jnp.float32),
                   jax.ShapeDtypeStruct((N_NODES, 1), jnp.float32)),
    )(p, h)


def _tc_tp2(ev, ef, od, fc2_w1, fc2_b1, A2, B2):
    grid = N_EDGES // EB

    def body(ev_ref, ef_ref, od_ref, w1_ref, b1_ref, a_ref, bb_ref,
             rk_ref, tu_ref, m1_ref, m2_ref, ones3_ref, md1_ref, md2_ref,
             ones48_ref, yp_ref, msk_ref, o_ref):
        shD = _sh_expand(ev_ref[...], md1_ref[...], md2_ref[...],
                         ones48_ref[...], m1_ref[...], m2_ref[...],
                         ones3_ref[...])
        f2 = _softplus(_dot(ef_ref[...], w1_ref[...]) + b1_ref[...])
        od_b = jnp.where(msk_ref[...] > 0.0, od_ref[...], 0.0)
        yv = _dot(od_b * shD, yp_ref[...])
        g = _dot(f2, rk_ref[...]) * _dot(yv, tu_ref[...])
        o_ref[...] = C2 * (_dot(g, a_ref[...]) + _dot(yv, bb_ref[...]))

    full = lambda shape: pl.BlockSpec(shape, lambda i: (0, 0))
    blk = lambda d: pl.BlockSpec((EB, d), lambda i: (i, 0))
    return pl.pallas_call(
        body,
        grid=(grid,),
        in_specs=[blk(3), blk(16), blk(128), full((16, 16)), full((1, 16)),
                  full((384, 128)), full((24, 128)), full((16, 384)),
                  full((24, 384)), full((3, 6)), full((3, 6)), full((3, 1)),
                  full((3, 128)), full((6, 128)), full((1, 128)),
                  full((128, 24)), full((1, 128))],
        out_specs=blk(128),
        out_shape=jax.ShapeDtypeStruct((N_EDGES, 128), jnp.float32),
    )(ev, ef, od, fc2_w1, fc2_b1.reshape(1, 16),
      jnp.pad(A2, ((0, 0), (0, 112))), jnp.pad(B2, ((0, 0), (0, 112))),
      jnp.asarray(_RK2), jnp.asarray(_TU2),
      jnp.asarray(_SH['M1']), jnp.asarray(_SH['M2']), jnp.asarray(_SH['ONES3']),
      jnp.asarray(_SH['MD1']), jnp.asarray(_SH['MD2']),
      jnp.asarray(_SH['ONES16_48']), jnp.asarray(_SH['YP']),
      jnp.asarray(_SH['MASK48']))


def _tc_final(q, cntc, bn_gamma, bn_beta, lin2_w, lin2_b, skip):
    def body(q_ref, c_ref, g_ref, b_ref, w_ref, wb_ref, s_ref, o_ref):
        o2 = (q_ref[0] + q_ref[1])[:, :16] / c_ref[...]
        mean = jnp.mean(o2, axis=0, keepdims=True)
        var = jnp.mean((o2 - mean) ** 2, axis=0, keepdims=True)
        xn = (o2 - mean) / jnp.sqrt(var + 1e-5) * g_ref[...] + b_ref[...]
        h3 = _softplus(_dot(_softplus(xn), w_ref[...]) + wb_ref[...])
        o_ref[...] = h3 + s_ref[...]

    return pl.pallas_call(
        body,
        in_specs=[pl.BlockSpec((2, N_NODES, 128), lambda: (0, 0, 0)),
                  pl.BlockSpec((N_NODES, 1), lambda: (0, 0)),
                  pl.BlockSpec((1, 16), lambda: (0, 0)),
                  pl.BlockSpec((1, 16), lambda: (0, 0)),
                  pl.BlockSpec((16, 128), lambda: (0, 0)),
                  pl.BlockSpec((1, 128), lambda: (0, 0)),
                  pl.BlockSpec((N_NODES, 128), lambda: (0, 0))],
        out_shape=jax.ShapeDtypeStruct((N_NODES, 128), jnp.float32),
    )(q, cntc, bn_gamma.reshape(1, 16), bn_beta.reshape(1, 16),
      lin2_w, lin2_b.reshape(1, 128), skip)


def kernel(edge_vec, node_feature, edge_index, edge_feature, lin1_w, lin1_b,
           fc1_w1, fc1_b1, fc1_w2, fc1_b2, fc2_w1, fc2_b1, fc2_w2, fc2_b2,
           bn_gamma, bn_beta, lin2_w, lin2_b):
    A1 = fc1_w2[:, _COL1].reshape(256, 24)
    B1 = fc1_b2[_COL1]
    A2 = fc2_w2[:, _COL2].reshape(384, 16)
    B2 = fc2_b2[_COL2]

    pe = IDX_PAD - N_EDGES
    src2d = jnp.pad(edge_index[0], (0, pe)).reshape(IDX_PAD // GRP, GRP)
    dst2d = jnp.pad(edge_index[1], (0, pe)).reshape(IDX_PAD // GRP, GRP)

    h = _tc_h(node_feature, lin1_w, lin1_b)
    hd = _sc_gather(h, dst2d, 16, gblk=5)
    tp1e = _tc_tp1(edge_vec, edge_feature, hd, fc1_w1, fc1_b1, A1, B1)
    p = _sc_scatter_add(tp1e, src2d, 64, gblk=2)
    out1, cntc = _tc_out1(p, h)
    od = _sc_gather(out1, dst2d, 48, gblk=5)
    tp2 = _tc_tp2(edge_vec, edge_feature, od, fc2_w1, fc2_b1, A2, B2)
    q = _sc_scatter_add(tp2, src2d, 16, gblk=5)
    return _tc_final(q, cntc, bn_gamma, bn_beta, lin2_w, lin2_b,
                     node_feature)


# speedup vs baseline: 5.2234x; 1.0007x over previous
import functools

import jax
import jax.numpy as jnp
import numpy as np
from jax import lax
from jax.experimental import pallas as pl
from jax.experimental.pallas import tpu as pltpu
from jax.experimental.pallas import tpu_sc as plsc

N_NODES = 10000
N_EDGES = 160000
NW = 32
GRP = 128
IDX_PAD = 163840
EB = 4000
S3 = float(np.sqrt(3.0))
S5 = float(np.sqrt(5.0))
S15 = float(np.sqrt(15.0))
C1 = 1.0 / float(np.sqrt(16.0))
C2 = 1.0 / float(np.sqrt(24.0))


def _softplus(x):
    m = jnp.maximum(x, 0.0)
    return m + jnp.log(jnp.exp(x - m) + jnp.exp(-m))


def _dot(a, b):
    return jnp.dot(a, b, preferred_element_type=jnp.float32)


def _np_consts():
    col1 = np.zeros((16, 24), dtype=np.int32)
    for u in range(16):
        for j in range(24):
            if j < 16:
                col1[u, j] = u * 16 + j
            elif j < 20:
                col1[u, j] = 256 + u * 4 + (j - 16)
            else:
                col1[u, j] = 320 + u * 4 + (j - 20)
    col2 = np.zeros((24, 16), dtype=np.int32)
    for u in range(24):
        for w in range(16):
            if u < 16:
                col2[u, w] = u * 16 + w
            elif u < 20:
                col2[u, w] = 256 + (u - 16) * 16 + w
            else:
                col2[u, w] = 320 + (u - 20) * 16 + w
    RK = np.repeat(np.eye(16, dtype=np.float32), 16, axis=1)
    TU = np.tile(np.eye(16, dtype=np.float32), (1, 16))
    RK2 = np.repeat(np.eye(16, dtype=np.float32), 24, axis=1)
    TU2 = np.tile(np.eye(24, dtype=np.float32), (1, 16))
    M1 = np.zeros((3, 6), np.float32)
    M2 = np.zeros((3, 6), np.float32)
    first = [0, 0, 1, 0, 2, 1]
    second = [2, 1, 1, 0, 2, 2]
    for c in range(6):
        M1[first[c], c] = 1.0
        M2[second[c], c] = 1.0
    C2m = np.zeros((6, 5), np.float32)
    C2m[0, 0] = S15
    C2m[1, 1] = S15
    C2m[2, 2] = S5
    C2m[3, 2] = -0.5 * S5
    C2m[4, 2] = -0.5 * S5
    C2m[5, 3] = S15
    C2m[4, 4] = 0.5 * S15
    C2m[3, 4] = -0.5 * S15
    TEXP = np.zeros((24, 128), np.float32)
    for j in range(16):
        TEXP[j, j] = 1.0
    for v in range(4):
        for m in range(3):
            TEXP[16 + v, 16 + 3 * v + m] = 1.0
        for c in range(5):
            TEXP[20 + v, 28 + 5 * v + c] = 1.0
    MQ1p = np.zeros((3, 128), np.float32)
    MQ2p = np.zeros((6, 128), np.float32)
    for v in range(4):
        for m in range(3):
            MQ1p[m, 16 + 3 * v + m] = S3
        for c in range(5):
            MQ2p[:, 28 + 5 * v + c] = C2m[:, c]
    ONES64 = np.zeros((1, 128), np.float32)
    ONES64[0, :16] = 1.0
    ONES64[0, 48] = 1.0
    ONE48 = np.zeros((1, 128), np.float32)
    ONE48[0, 48] = 1.0
    MD1 = np.zeros((3, 128), np.float32)
    MD2 = np.zeros((6, 128), np.float32)
    for v in range(4):
        for m in range(3):
            MD1[m, 16 + 3 * v + m] = S3
        for c in range(5):
            MD2[:, 28 + 5 * v + c] = C2m[:, c]
    ONES16_48 = np.zeros((1, 128), np.float32)
    ONES16_48[0, :16] = 1.0
    YP = np.zeros((128, 24), np.float32)
    for j in range(16):
        YP[j, j] = 1.0
    for v in range(4):
        for m in range(3):
            YP[16 + 3 * v + m, 16 + v] = 1.0 / S3
        for c in range(5):
            YP[28 + 5 * v + c, 20 + v] = 1.0 / S5
    TU128 = np.zeros((128, 256), np.float32)
    TU128[:16] = TU
    MASK16 = np.zeros((1, 128), np.float32)
    MASK16[0, :16] = 1.0
    MASK48 = np.zeros((1, 128), np.float32)
    MASK48[0, :48] = 1.0
    sh = dict(M1=M1, M2=M2, TEXP=TEXP, MQ1p=MQ1p, MQ2p=MQ2p, ONES64=ONES64,
              ONE48=ONE48, MD1=MD1, MD2=MD2, ONES16_48=ONES16_48, YP=YP,
              TU128=TU128, MASK16=MASK16, MASK48=MASK48,
              ONES3=np.ones((3, 1), np.float32))
    return col1, col2, RK, TU, RK2, TU2, sh


_COL1, _COL2, _RK, _TU, _RK2, _TU2, _SH = _np_consts()


def _sc_gather(table, idx2d, d, gblk):
    ch = GRP * gblk
    cpw = (IDX_PAD // ch) // NW
    real = N_EDGES // ch
    assert real * ch == N_EDGES
    mesh = plsc.VectorSubcoreMesh(core_axis_name="c", subcore_axis_name="s")

    @functools.partial(
        pl.kernel, mesh=mesh,
        out_type=jax.ShapeDtypeStruct((N_EDGES, 128), jnp.float32),
        compiler_params=pltpu.CompilerParams(use_tc_tiling_on_sc=False),
        scratch_types=[
            pltpu.VMEM((2, gblk, GRP), jnp.int32),
            pltpu.VMEM((2, ch, d), jnp.float32),
            pltpu.SemaphoreType.DMA,
            pltpu.SemaphoreType.DMA,
            pltpu.SemaphoreType.DMA,
            pltpu.SemaphoreType.DMA,
            pltpu.SemaphoreType.DMA,
        ],
    )
    def k(table_hbm, idx_hbm, out_hbm, idx_v, rows_v,
          sem_i0, sem_i1, sem_g, sem_s0, sem_s1):
        wid = lax.axis_index("s") * 2 + lax.axis_index("c")
        sem_i = (sem_i0, sem_i1)
        sem_s = (sem_s0, sem_s1)

        def pred(c):
            return wid * cpw + c < real

        def idx_copy(c):
            g = (wid * cpw + c) * gblk
            return pltpu.make_async_copy(
                idx_hbm.at[pl.ds(g, gblk)], idx_v.at[c % 2], sem_i[c % 2])

        def out_copy(c):
            eoff = (wid * cpw + c) * ch
            return pltpu.make_async_copy(
                rows_v.at[c % 2],
                out_hbm.at[pl.ds(eoff, ch), pl.ds(0, d)], sem_s[c % 2])

        @pl.when(pred(0))
        def _():
            idx_copy(0).start()

        for c in range(cpw):
            b = c % 2
            if c + 1 < cpw:
                @pl.when(pred(c + 1))
                def _(c=c):
                    idx_copy(c + 1).start()

            if c >= 2:
                @pl.when(pred(c - 2))
                def _(c=c):
                    out_copy(c - 2).wait()

            @pl.when(pred(c))
            def _(c=c, b=b):
                idx_copy(c).wait()
                gathers = [
                    pltpu.make_async_copy(
                        table_hbm.at[idx_v.at[b, j]],
                        rows_v.at[b, pl.ds(j * GRP, GRP)], sem_g)
                    for j in range(gblk)
                ]
                for gcp in gathers:
                    gcp.start()
                for gcp in gathers:
                    gcp.wait()
                out_copy(c).start()

        for c in range(max(cpw - 2, 0), cpw):
            @pl.when(pred(c))
            def _(c=c):
                out_copy(c).wait()

    return k(table, idx2d)


def _sc_scatter_add(vals, idx2d, d, gblk):
    ch = GRP * gblk
    cpw = (IDX_PAD // ch) // NW
    real = N_EDGES // ch
    assert real * ch == N_EDGES
    rows_pt = N_NODES // 16
    mesh = plsc.VectorSubcoreMesh(core_axis_name="c", subcore_axis_name="s")
    zeros = jnp.zeros((N_NODES, d), jnp.float32)

    @functools.partial(
        pl.kernel, mesh=mesh,
        out_type=jax.ShapeDtypeStruct((2, N_NODES, 128), jnp.float32),
        compiler_params=pltpu.CompilerParams(use_tc_tiling_on_sc=False),
        scratch_types=[
            pltpu.VMEM((3, gblk, GRP), jnp.int32),
            pltpu.VMEM((3, ch, d), jnp.float32),
            pltpu.VMEM_SHARED((N_NODES, d), jnp.float32),
            pltpu.SemaphoreType.DMA,
            pltpu.SemaphoreType.DMA,
            pltpu.SemaphoreType.DMA,
            pltpu.SemaphoreType.DMA,
            pltpu.SemaphoreType.DMA,
            pltpu.SemaphoreType.DMA,
            pltpu.SemaphoreType.DMA,
        ],
    )
    def k(vals_hbm, idx_hbm, zero_hbm, out_hbm, idx_v, rows_v, acc_sh,
          sem_i0, sem_i1, sem_i2, sem_v0, sem_v1, sem_v2, sem_sc):
        cid = lax.axis_index("c")
        sid = lax.axis_index("s")
        wid = sid * 2 + cid
        r0 = sid * rows_pt
        pltpu.sync_copy(zero_hbm.at[pl.ds(r0, rows_pt)],
                        acc_sh.at[pl.ds(r0, rows_pt)])
        plsc.subcore_barrier()
        sem_i = (sem_i0, sem_i1, sem_i2)
        sem_v = (sem_v0, sem_v1, sem_v2)

        def pred(c):
            return wid * cpw + c < real

        def idx_copy(c):
            g = (wid * cpw + c) * gblk
            return pltpu.make_async_copy(
                idx_hbm.at[pl.ds(g, gblk)], idx_v.at[c % 3], sem_i[c % 3])

        def val_copy(c):
            eoff = (wid * cpw + c) * ch
            return pltpu.make_async_copy(
                vals_hbm.at[pl.ds(eoff, ch), pl.ds(0, d)],
                rows_v.at[c % 3], sem_v[c % 3])

        def scatters(c):
            b = c % 3
            return [
                pltpu.make_async_copy(
                    rows_v.at[b, pl.ds(j * GRP, GRP)],
                    acc_sh.at[idx_v.at[b, j]], sem_sc)
                for j in range(gblk)
            ]

        @pl.when(pred(0))
        def _():
            idx_copy(0).start()
            val_copy(0).start()

        if cpw > 1:
            @pl.when(pred(1))
            def _():
                idx_copy(1).start()
                val_copy(1).start()

        for c in range(cpw):
            if c >= 2:
                @pl.when(pred(c - 2))
                def _(c=c):
                    for scp in scatters(c - 2):
                        scp.wait()
            if c >= 1 and c + 1 < cpw:
                @pl.when(pred(c + 1))
                def _(c=c):
                    idx_copy(c + 1).start()
                    val_copy(c + 1).start()

            @pl.when(pred(c))
            def _(c=c):
                idx_copy(c).wait()
                val_copy(c).wait()
                for scp in scatters(c):
                    scp.start(add=True)

        for c in range(max(cpw - 2, 0), cpw):
            @pl.when(pred(c))
            def _(c=c):
                for scp in scatters(c):
                    scp.wait()
        plsc.subcore_barrier()
        pltpu.sync_copy(acc_sh.at[pl.ds(r0, rows_pt)],
                        out_hbm.at[cid, pl.ds(r0, rows_pt), pl.ds(0, d)])

    return k(vals, idx2d, zeros)


def _tc_h(node_feature, lin1_w, lin1_b):
    def body(nf, w, b, o):
        o[...] = _dot(nf[...], w[...]) + b[...]

    return pl.pallas_call(
        body,
        out_shape=jax.ShapeDtypeStruct((N_NODES, 16), jnp.float32),
    )(node_feature, lin1_w, lin1_b.reshape(1, 16))


def _sh_expand(ev, mq1, mq2, ones_row, m1, m2, ones3):
    ss = _dot(ev * ev, ones3)
    rr = jnp.where(ss == 0.0, 1.0, ss)
    rinv = lax.rsqrt(rr)
    r2inv = 1.0 / rr
    evn = ev * rinv
    praw = (_dot(ev, m1) * _dot(ev, m2)) * r2inv
    return ones_row + _dot(evn, mq1) + _dot(praw, mq2)


def _tc_tp1(ev, ef, hd, fc1_w1, fc1_b1, A1, B1):
    grid = N_EDGES // EB

    def body(ev_ref, ef_ref, hd_ref, w1_ref, b1_ref, a_ref, bb_ref,
             rk_ref, tu_ref, m1_ref, m2_ref, ones3_ref, mq1_ref, mq2_ref,
             ones64_ref, one48_ref, texp_ref, msk_ref, o_ref):
        shE = _sh_expand(ev_ref[...], mq1_ref[...], mq2_ref[...],
                         ones64_ref[...], m1_ref[...], m2_ref[...],
                         ones3_ref[...])
        f1 = _softplus(_dot(ef_ref[...], w1_ref[...]) + b1_ref[...])
        hd_b = jnp.where(msk_ref[...] > 0.0, hd_ref[...], 0.0)
        g = _dot(f1, rk_ref[...]) * _dot(hd_b, tu_ref[...])
        t = C1 * (_dot(g, a_ref[...]) + _dot(hd_b, bb_ref[...]))
        o_ref[...] = (_dot(t, texp_ref[...]) + one48_ref[...]) * shE

    full = lambda shape: pl.BlockSpec(shape, lambda i: (0, 0))
    blk = lambda d: pl.BlockSpec((EB, d), lambda i: (i, 0))
    return pl.pallas_call(
        body,
        grid=(grid,),
        in_specs=[blk(3), blk(16), blk(128), full((16, 16)), full((1, 16)),
                  full((256, 24)), full((128, 24)), full((16, 256)),
                  full((128, 256)), full((3, 6)), full((3, 6)), full((3, 1)),
                  full((3, 128)), full((6, 128)), full((1, 128)),
                  full((1, 128)), full((24, 128)), full((1, 128))],
        out_specs=blk(128),
        out_shape=jax.ShapeDtypeStruct((N_EDGES, 128), jnp.float32),
    )(ev, ef, hd, fc1_w1, fc1_b1.reshape(1, 16), A1,
      jnp.pad(B1, ((0, 112), (0, 0))),
      jnp.asarray(_RK), jnp.asarray(_SH['TU128']),
      jnp.asarray(_SH['M1']), jnp.asarray(_SH['M2']), jnp.asarray(_SH['ONES3']),
      jnp.asarray(_SH['MQ1p']), jnp.asarray(_SH['MQ2p']),
      jnp.asarray(_SH['ONES64']), jnp.asarray(_SH['ONE48']),
      jnp.asarray(_SH['TEXP']), jnp.asarray(_SH['MASK16']))


def _tc_out1(p, h):
    def body(p_ref, h_ref, o_ref, c_ref):
        s = p_ref[0] + p_ref[1]
        cnt = jnp.maximum(s[:, 48:49], 1.0)
        o = s[:, :48] / cnt
        o = o + jnp.concatenate(
            [h_ref[...], jnp.zeros((N_NODES, 32), jnp.float32)], axis=1)
        o_ref[...] = o
        c_ref[...] = cnt

    return pl.pallas_call(
        body,
        in_specs=[pl.BlockSpec((2, N_NODES, 128), lambda: (0, 0, 0)),
                  pl.BlockSpec((N_NODES, 16), lambda: (0, 0))],
        out_shape=(jax.ShapeDtypeStruct((N_NODES, 48), jnp.float32),
                   jax.ShapeDtypeStruct((N_NODES, 1), jnp.float32)),
    )(p, h)


def _tc_tp2(ev, ef, od, fc2_w1, fc2_b1, A2, B2):
    grid = N_EDGES // EB

    def body(ev_ref, ef_ref, od_ref, w1_ref, b1_ref, a_ref, bb_ref,
             rk_ref, tu_ref, m1_ref, m2_ref, ones3_ref, md1_ref, md2_ref,
             ones48_ref, yp_ref, msk_ref, o_ref):
        shD = _sh_expand(ev_ref[...], md1_ref[...], md2_ref[...],
                         ones48_ref[...], m1_ref[...], m2_ref[...],
                         ones3_ref[...])
        f2 = _softplus(_dot(ef_ref[...], w1_ref[...]) + b1_ref[...])
        od_b = jnp.where(msk_ref[...] > 0.0, od_ref[...], 0.0)
        yv = _dot(od_b * shD, yp_ref[...])
        g = _dot(f2, rk_ref[...]) * _dot(yv, tu_ref[...])
        o_ref[...] = C2 * (_dot(g, a_ref[...]) + _dot(yv, bb_ref[...]))

    full = lambda shape: pl.BlockSpec(shape, lambda i: (0, 0))
    blk = lambda d: pl.BlockSpec((EB, d), lambda i: (i, 0))
    return pl.pallas_call(
        body,
        grid=(grid,),
        in_specs=[blk(3), blk(16), blk(128), full((16, 16)), full((1, 16)),
                  full((384, 128)), full((24, 128)), full((16, 384)),
                  full((24, 384)), full((3, 6)), full((3, 6)), full((3, 1)),
                  full((3, 128)), full((6, 128)), full((1, 128)),
                  full((128, 24)), full((1, 128))],
        out_specs=blk(128),
        out_shape=jax.ShapeDtypeStruct((N_EDGES, 128), jnp.float32),
    )(ev, ef, od, fc2_w1, fc2_b1.reshape(1, 16),
      jnp.pad(A2, ((0, 0), (0, 112))), jnp.pad(B2, ((0, 0), (0, 112))),
      jnp.asarray(_RK2), jnp.asarray(_TU2),
      jnp.asarray(_SH['M1']), jnp.asarray(_SH['M2']), jnp.asarray(_SH['ONES3']),
      jnp.asarray(_SH['MD1']), jnp.asarray(_SH['MD2']),
      jnp.asarray(_SH['ONES16_48']), jnp.asarray(_SH['YP']),
      jnp.asarray(_SH['MASK48']))


def _tc_final(q, cntc, bn_gamma, bn_beta, lin2_w, lin2_b, skip):
    def body(q_ref, c_ref, g_ref, b_ref, w_ref, wb_ref, s_ref, o_ref):
        o2 = (q_ref[0] + q_ref[1])[:, :16] / c_ref[...]
        mean = jnp.mean(o2, axis=0, keepdims=True)
        var = jnp.mean((o2 - mean) ** 2, axis=0, keepdims=True)
        xn = (o2 - mean) / jnp.sqrt(var + 1e-5) * g_ref[...] + b_ref[...]
        h3 = _softplus(_dot(_softplus(xn), w_ref[...]) + wb_ref[...])
        o_ref[...] = h3 + s_ref[...]

    return pl.pallas_call(
        body,
        in_specs=[pl.BlockSpec((2, N_NODES, 128), lambda: (0, 0, 0)),
                  pl.BlockSpec((N_NODES, 1), lambda: (0, 0)),
                  pl.BlockSpec((1, 16), lambda: (0, 0)),
                  pl.BlockSpec((1, 16), lambda: (0, 0)),
                  pl.BlockSpec((16, 128), lambda: (0, 0)),
                  pl.BlockSpec((1, 128), lambda: (0, 0)),
                  pl.BlockSpec((N_NODES, 128), lambda: (0, 0))],
        out_shape=jax.ShapeDtypeStruct((N_NODES, 128), jnp.float32),
    )(q, cntc, bn_gamma.reshape(1, 16), bn_beta.reshape(1, 16),
      lin2_w, lin2_b.reshape(1, 128), skip)


def kernel(edge_vec, node_feature, edge_index, edge_feature, lin1_w, lin1_b,
           fc1_w1, fc1_b1, fc1_w2, fc1_b2, fc2_w1, fc2_b1, fc2_w2, fc2_b2,
           bn_gamma, bn_beta, lin2_w, lin2_b):
    A1 = fc1_w2[:, _COL1].reshape(256, 24)
    B1 = fc1_b2[_COL1]
    A2 = fc2_w2[:, _COL2].reshape(384, 16)
    B2 = fc2_b2[_COL2]

    pe = IDX_PAD - N_EDGES
    src2d = jnp.pad(edge_index[0], (0, pe)).reshape(IDX_PAD // GRP, GRP)
    dst2d = jnp.pad(edge_index[1], (0, pe)).reshape(IDX_PAD // GRP, GRP)

    h = _tc_h(node_feature, lin1_w, lin1_b)
    hd = _sc_gather(h, dst2d, 16, gblk=5)
    tp1e = _tc_tp1(edge_vec, edge_feature, hd, fc1_w1, fc1_b1, A1, B1)
    p = _sc_scatter_add(tp1e, src2d, 64, gblk=2)
    out1, cntc = _tc_out1(p, h)
    od = _sc_gather(out1, dst2d, 48, gblk=5)
    tp2 = _tc_tp2(edge_vec, edge_feature, od, fc2_w1, fc2_b1, A2, B2)
    q = _sc_scatter_add(tp2, src2d, 16, gblk=5)
    return _tc_final(q, cntc, bn_gamma, bn_beta, lin2_w, lin2_b,
                     node_feature)


# speedup vs baseline: 5.3464x; 1.0235x over previous
import functools

import jax
import jax.numpy as jnp
import numpy as np
from jax import lax
from jax.experimental import pallas as pl
from jax.experimental.pallas import tpu as pltpu
from jax.experimental.pallas import tpu_sc as plsc

N_NODES = 10000
N_EDGES = 160000
NW = 32
GRP = 128
IDX_PAD = 163840
EB = 3200
S3 = float(np.sqrt(3.0))
S5 = float(np.sqrt(5.0))
S15 = float(np.sqrt(15.0))
C1 = 1.0 / float(np.sqrt(16.0))
C2 = 1.0 / float(np.sqrt(24.0))


def _softplus(x):
    m = jnp.maximum(x, 0.0)
    return m + jnp.log(jnp.exp(x - m) + jnp.exp(-m))


def _dot(a, b):
    return jnp.dot(a, b, preferred_element_type=jnp.float32)


def _dotT(a, b):
    return lax.dot_general(a, b, (((0,), (0,)), ((), ())),
                           preferred_element_type=jnp.float32)


def _np_consts():
    col1 = np.zeros((16, 24), dtype=np.int32)
    for u in range(16):
        for j in range(24):
            if j < 16:
                col1[u, j] = u * 16 + j
            elif j < 20:
                col1[u, j] = 256 + u * 4 + (j - 16)
            else:
                col1[u, j] = 320 + u * 4 + (j - 20)
    col2 = np.zeros((24, 16), dtype=np.int32)
    for u in range(24):
        for w in range(16):
            if u < 16:
                col2[u, w] = u * 16 + w
            elif u < 20:
                col2[u, w] = 256 + (u - 16) * 16 + w
            else:
                col2[u, w] = 320 + (u - 20) * 16 + w
    RK = np.repeat(np.eye(16, dtype=np.float32), 16, axis=1)
    TU = np.tile(np.eye(16, dtype=np.float32), (1, 16))
    RK2 = np.repeat(np.eye(16, dtype=np.float32), 24, axis=1)
    TU2 = np.tile(np.eye(24, dtype=np.float32), (1, 16))
    M1 = np.zeros((3, 6), np.float32)
    M2 = np.zeros((3, 6), np.float32)
    first = [0, 0, 1, 0, 2, 1]
    second = [2, 1, 1, 0, 2, 2]
    for c in range(6):
        M1[first[c], c] = 1.0
        M2[second[c], c] = 1.0
    C2m = np.zeros((6, 5), np.float32)
    C2m[0, 0] = S15
    C2m[1, 1] = S15
    C2m[2, 2] = S5
    C2m[3, 2] = -0.5 * S5
    C2m[4, 2] = -0.5 * S5
    C2m[5, 3] = S15
    C2m[4, 4] = 0.5 * S15
    C2m[3, 4] = -0.5 * S15
    TEXP = np.zeros((24, 128), np.float32)
    for j in range(16):
        TEXP[j, j] = 1.0
    for v in range(4):
        for m in range(3):
            TEXP[16 + v, 16 + 3 * v + m] = 1.0
        for c in range(5):
            TEXP[20 + v, 28 + 5 * v + c] = 1.0
    MQ1p = np.zeros((3, 128), np.float32)
    MQ2p = np.zeros((6, 128), np.float32)
    for v in range(4):
        for m in range(3):
            MQ1p[m, 16 + 3 * v + m] = S3
        for c in range(5):
            MQ2p[:, 28 + 5 * v + c] = C2m[:, c]
    ONES64 = np.zeros((1, 128), np.float32)
    ONES64[0, :16] = 1.0
    ONES64[0, 48] = 1.0
    ONE48 = np.zeros((1, 128), np.float32)
    ONE48[0, 48] = 1.0
    MD1 = np.zeros((3, 128), np.float32)
    MD2 = np.zeros((6, 128), np.float32)
    for v in range(4):
        for m in range(3):
            MD1[m, 16 + 3 * v + m] = S3
        for c in range(5):
            MD2[:, 28 + 5 * v + c] = C2m[:, c]
    ONES16_48 = np.zeros((1, 128), np.float32)
    ONES16_48[0, :16] = 1.0
    YP = np.zeros((128, 24), np.float32)
    for j in range(16):
        YP[j, j] = 1.0
    for v in range(4):
        for m in range(3):
            YP[16 + 3 * v + m, 16 + v] = 1.0 / S3
        for c in range(5):
            YP[28 + 5 * v + c, 20 + v] = 1.0 / S5
    TU128 = np.zeros((128, 256), np.float32)
    TU128[:16] = TU
    MASK16 = np.zeros((1, 128), np.float32)
    MASK16[0, :16] = 1.0
    MASK48 = np.zeros((1, 128), np.float32)
    MASK48[0, :48] = 1.0
    sh = dict(M1=M1, M2=M2, TEXP=TEXP, MQ1p=MQ1p, MQ2p=MQ2p, ONES64=ONES64,
              ONE48=ONE48, MD1=MD1, MD2=MD2, ONES16_48=ONES16_48, YP=YP,
              TU128=TU128, MASK16=MASK16, MASK48=MASK48,
              ONES3=np.ones((3, 1), np.float32))
    return col1, col2, RK, TU, RK2, TU2, sh


_COL1, _COL2, _RK, _TU, _RK2, _TU2, _SH = _np_consts()


def _sc_gather(table, idx2d, d, gblk):
    ch = GRP * gblk
    cpw = (IDX_PAD // ch) // NW
    real = N_EDGES // ch
    assert real * ch == N_EDGES
    mesh = plsc.VectorSubcoreMesh(core_axis_name="c", subcore_axis_name="s")

    @functools.partial(
        pl.kernel, mesh=mesh,
        out_type=jax.ShapeDtypeStruct((N_EDGES, 128), jnp.float32),
        compiler_params=pltpu.CompilerParams(use_tc_tiling_on_sc=False),
        scratch_types=[
            pltpu.VMEM((2, gblk, GRP), jnp.int32),
            pltpu.VMEM((2, ch, d), jnp.float32),
            pltpu.SemaphoreType.DMA,
            pltpu.SemaphoreType.DMA,
            pltpu.SemaphoreType.DMA,
            pltpu.SemaphoreType.DMA,
            pltpu.SemaphoreType.DMA,
        ],
    )
    def k(table_hbm, idx_hbm, out_hbm, idx_v, rows_v,
          sem_i0, sem_i1, sem_g, sem_s0, sem_s1):
        wid = lax.axis_index("s") * 2 + lax.axis_index("c")
        sem_i = (sem_i0, sem_i1)
        sem_s = (sem_s0, sem_s1)

        def pred(c):
            return wid * cpw + c < real

        def idx_copy(c):
            g = (wid * cpw + c) * gblk
            return pltpu.make_async_copy(
                idx_hbm.at[pl.ds(g, gblk)], idx_v.at[c % 2], sem_i[c % 2])

        def out_copy(c):
            eoff = (wid * cpw + c) * ch
            return pltpu.make_async_copy(
                rows_v.at[c % 2],
                out_hbm.at[pl.ds(eoff, ch), pl.ds(0, d)], sem_s[c % 2])

        @pl.when(pred(0))
        def _():
            idx_copy(0).start()

        for c in range(cpw):
            b = c % 2
            if c + 1 < cpw:
                @pl.when(pred(c + 1))
                def _(c=c):
                    idx_copy(c + 1).start()

            if c >= 2:
                @pl.when(pred(c - 2))
                def _(c=c):
                    out_copy(c - 2).wait()

            @pl.when(pred(c))
            def _(c=c, b=b):
                idx_copy(c).wait()
                gathers = [
                    pltpu.make_async_copy(
                        table_hbm.at[idx_v.at[b, j]],
                        rows_v.at[b, pl.ds(j * GRP, GRP)], sem_g)
                    for j in range(gblk)
                ]
                for gcp in gathers:
                    gcp.start()
                for gcp in gathers:
                    gcp.wait()
                out_copy(c).start()

        for c in range(max(cpw - 2, 0), cpw):
            @pl.when(pred(c))
            def _(c=c):
                out_copy(c).wait()

    return k(table, idx2d)


def _sc_scatter_add(vals, idx2d, d, gblk):
    ch = GRP * gblk
    cpw = (IDX_PAD // ch) // NW
    real = N_EDGES // ch
    assert real * ch == N_EDGES
    rows_pt = N_NODES // 16
    mesh = plsc.VectorSubcoreMesh(core_axis_name="c", subcore_axis_name="s")
    zeros = jnp.zeros((N_NODES, d), jnp.float32)

    @functools.partial(
        pl.kernel, mesh=mesh,
        out_type=jax.ShapeDtypeStruct((2, N_NODES, 128), jnp.float32),
        compiler_params=pltpu.CompilerParams(use_tc_tiling_on_sc=False),
        scratch_types=[
            pltpu.VMEM((3, gblk, GRP), jnp.int32),
            pltpu.VMEM((3, ch, d), jnp.float32),
            pltpu.VMEM_SHARED((N_NODES, d), jnp.float32),
            pltpu.SemaphoreType.DMA,
            pltpu.SemaphoreType.DMA,
            pltpu.SemaphoreType.DMA,
            pltpu.SemaphoreType.DMA,
            pltpu.SemaphoreType.DMA,
            pltpu.SemaphoreType.DMA,
            pltpu.SemaphoreType.DMA,
        ],
    )
    def k(vals_hbm, idx_hbm, zero_hbm, out_hbm, idx_v, rows_v, acc_sh,
          sem_i0, sem_i1, sem_i2, sem_v0, sem_v1, sem_v2, sem_sc):
        cid = lax.axis_index("c")
        sid = lax.axis_index("s")
        wid = sid * 2 + cid
        r0 = sid * rows_pt
        pltpu.sync_copy(zero_hbm.at[pl.ds(r0, rows_pt)],
                        acc_sh.at[pl.ds(r0, rows_pt)])
        plsc.subcore_barrier()
        sem_i = (sem_i0, sem_i1, sem_i2)
        sem_v = (sem_v0, sem_v1, sem_v2)

        def pred(c):
            return wid * cpw + c < real

        def idx_copy(c):
            g = (wid * cpw + c) * gblk
            return pltpu.make_async_copy(
                idx_hbm.at[pl.ds(g, gblk)], idx_v.at[c % 3], sem_i[c % 3])

        def val_copy(c):
            eoff = (wid * cpw + c) * ch
            return pltpu.make_async_copy(
                vals_hbm.at[pl.ds(eoff, ch), pl.ds(0, d)],
                rows_v.at[c % 3], sem_v[c % 3])

        def scatters(c):
            b = c % 3
            return [
                pltpu.make_async_copy(
                    rows_v.at[b, pl.ds(j * GRP, GRP)],
                    acc_sh.at[idx_v.at[b, j]], sem_sc)
                for j in range(gblk)
            ]

        @pl.when(pred(0))
        def _():
            idx_copy(0).start()
            val_copy(0).start()

        if cpw > 1:
            @pl.when(pred(1))
            def _():
                idx_copy(1).start()
                val_copy(1).start()

        for c in range(cpw):
            if c >= 2:
                @pl.when(pred(c - 2))
                def _(c=c):
                    for scp in scatters(c - 2):
                        scp.wait()
            if c >= 1 and c + 1 < cpw:
                @pl.when(pred(c + 1))
                def _(c=c):
                    idx_copy(c + 1).start()
                    val_copy(c + 1).start()

            @pl.when(pred(c))
            def _(c=c):
                idx_copy(c).wait()
                val_copy(c).wait()
                for scp in scatters(c):
                    scp.start(add=True)

        for c in range(max(cpw - 2, 0), cpw):
            @pl.when(pred(c))
            def _(c=c):
                for scp in scatters(c):
                    scp.wait()
        plsc.subcore_barrier()
        pltpu.sync_copy(acc_sh.at[pl.ds(r0, rows_pt)],
                        out_hbm.at[cid, pl.ds(r0, rows_pt), pl.ds(0, d)])

    return k(vals, idx2d, zeros)


def _tc_h(node_feature, lin1_w, lin1_b):
    def body(nf, w, b, o):
        o[...] = _dot(nf[...], w[...]) + b[...]

    return pl.pallas_call(
        body,
        out_shape=jax.ShapeDtypeStruct((N_NODES, 16), jnp.float32),
    )(node_feature, lin1_w, lin1_b.reshape(1, 16))


def _sh_expand(evT, mq1, mq2, ones_row, m1, m2, ones3):
    ss = _dotT(evT * evT, ones3)
    rr = jnp.where(ss == 0.0, 1.0, ss)
    rinv = lax.rsqrt(rr)
    r2inv = 1.0 / rr
    praw = (_dotT(evT, m1) * _dotT(evT, m2)) * r2inv
    return ones_row + _dotT(evT, mq1) * rinv + _dot(praw, mq2)


def _tc_tp1(ev, ef, hd, fc1_w1, fc1_b1, A1, B1):
    grid = N_EDGES // EB

    def body(ev_ref, ef_ref, hd_ref, w1_ref, b1_ref, a_ref, bb_ref,
             rk_ref, tu_ref, m1_ref, m2_ref, ones3_ref, mq1_ref, mq2_ref,
             ones64_ref, one48_ref, texp_ref, msk_ref, o_ref):
        shE = _sh_expand(ev_ref[...], mq1_ref[...], mq2_ref[...],
                         ones64_ref[...], m1_ref[...], m2_ref[...],
                         ones3_ref[...])
        f1 = _softplus(_dotT(ef_ref[...], w1_ref[...]) + b1_ref[...])
        hd_b = jnp.where(msk_ref[...] > 0.0, hd_ref[...], 0.0)
        g = _dot(f1, rk_ref[...]) * _dot(hd_b, tu_ref[...])
        t = C1 * (_dot(g, a_ref[...]) + _dot(hd_b, bb_ref[...]))
        o_ref[...] = (_dot(t, texp_ref[...]) + one48_ref[...]) * shE

    full = lambda shape: pl.BlockSpec(shape, lambda i: (0, 0))
    blk = lambda d: pl.BlockSpec((EB, d), lambda i: (i, 0))
    blkT = lambda r: pl.BlockSpec((r, EB), lambda i: (0, i))
    return pl.pallas_call(
        body,
        grid=(grid,),
        in_specs=[blkT(3), blkT(16), blk(128), full((16, 16)), full((1, 16)),
                  full((256, 24)), full((128, 24)), full((16, 256)),
                  full((128, 256)), full((3, 6)), full((3, 6)), full((3, 1)),
                  full((3, 128)), full((6, 128)), full((1, 128)),
                  full((1, 128)), full((24, 128)), full((1, 128))],
        out_specs=blk(128),
        out_shape=jax.ShapeDtypeStruct((N_EDGES, 128), jnp.float32),
    )(ev, ef, hd, fc1_w1, fc1_b1.reshape(1, 16), A1,
      jnp.pad(B1, ((0, 112), (0, 0))),
      jnp.asarray(_RK), jnp.asarray(_SH['TU128']),
      jnp.asarray(_SH['M1']), jnp.asarray(_SH['M2']), jnp.asarray(_SH['ONES3']),
      jnp.asarray(_SH['MQ1p']), jnp.asarray(_SH['MQ2p']),
      jnp.asarray(_SH['ONES64']), jnp.asarray(_SH['ONE48']),
      jnp.asarray(_SH['TEXP']), jnp.asarray(_SH['MASK16']))


def _tc_out1(p, h):
    def body(p_ref, h_ref, o_ref, c_ref):
        s = p_ref[0] + p_ref[1]
        cnt = jnp.maximum(s[:, 48:49], 1.0)
        o = s[:, :48] / cnt
        o = o + jnp.concatenate(
            [h_ref[...], jnp.zeros((N_NODES, 32), jnp.float32)], axis=1)
        o_ref[...] = o
        c_ref[...] = cnt

    return pl.pallas_call(
        body,
        in_specs=[pl.BlockSpec((2, N_NODES, 128), lambda: (0, 0, 0)),
                  pl.BlockSpec((N_NODES, 16), lambda: (0, 0))],
        out_shape=(jax.ShapeDtypeStruct((N_NODES, 48), jnp.float32),
                   jax.ShapeDtypeStruct((N_NODES, 1), jnp.float32)),
    )(p, h)


def _tc_tp2(ev, ef, od, fc2_w1, fc2_b1, A2, B2):
    grid = N_EDGES // EB

    def body(ev_ref, ef_ref, od_ref, w1_ref, b1_ref, a_ref, bb_ref,
             rk_ref, tu_ref, m1_ref, m2_ref, ones3_ref, md1_ref, md2_ref,
             ones48_ref, yp_ref, msk_ref, o_ref):
        shD = _sh_expand(ev_ref[...], md1_ref[...], md2_ref[...],
                         ones48_ref[...], m1_ref[...], m2_ref[...],
                         ones3_ref[...])
        f2 = _softplus(_dotT(ef_ref[...], w1_ref[...]) + b1_ref[...])
        od_b = jnp.where(msk_ref[...] > 0.0, od_ref[...], 0.0)
        yv = _dot(od_b * shD, yp_ref[...])
        g = _dot(f2, rk_ref[...]) * _dot(yv, tu_ref[...])
        o_ref[...] = C2 * (_dot(g, a_ref[...]) + _dot(yv, bb_ref[...]))

    full = lambda shape: pl.BlockSpec(shape, lambda i: (0, 0))
    blk = lambda d: pl.BlockSpec((EB, d), lambda i: (i, 0))
    blkT = lambda r: pl.BlockSpec((r, EB), lambda i: (0, i))
    return pl.pallas_call(
        body,
        grid=(grid,),
        in_specs=[blkT(3), blkT(16), blk(128), full((16, 16)), full((1, 16)),
                  full((384, 128)), full((24, 128)), full((16, 384)),
                  full((24, 384)), full((3, 6)), full((3, 6)), full((3, 1)),
                  full((3, 128)), full((6, 128)), full((1, 128)),
                  full((128, 24)), full((1, 128))],
        out_specs=blk(128),
        out_shape=jax.ShapeDtypeStruct((N_EDGES, 128), jnp.float32),
    )(ev, ef, od, fc2_w1, fc2_b1.reshape(1, 16),
      jnp.pad(A2, ((0, 0), (0, 112))), jnp.pad(B2, ((0, 0), (0, 112))),
      jnp.asarray(_RK2), jnp.asarray(_TU2),
      jnp.asarray(_SH['M1']), jnp.asarray(_SH['M2']), jnp.asarray(_SH['ONES3']),
      jnp.asarray(_SH['MD1']), jnp.asarray(_SH['MD2']),
      jnp.asarray(_SH['ONES16_48']), jnp.asarray(_SH['YP']),
      jnp.asarray(_SH['MASK48']))


def _tc_final(q, cntc, bn_gamma, bn_beta, lin2_w, lin2_b, skip):
    def body(q_ref, c_ref, g_ref, b_ref, w_ref, wb_ref, s_ref, o_ref):
        o2 = (q_ref[0] + q_ref[1])[:, :16] / c_ref[...]
        mean = jnp.mean(o2, axis=0, keepdims=True)
        var = jnp.mean((o2 - mean) ** 2, axis=0, keepdims=True)
        xn = (o2 - mean) / jnp.sqrt(var + 1e-5) * g_ref[...] + b_ref[...]
        h3 = _softplus(_dot(_softplus(xn), w_ref[...]) + wb_ref[...])
        o_ref[...] = h3 + s_ref[...]

    return pl.pallas_call(
        body,
        in_specs=[pl.BlockSpec((2, N_NODES, 128), lambda: (0, 0, 0)),
                  pl.BlockSpec((N_NODES, 1), lambda: (0, 0)),
                  pl.BlockSpec((1, 16), lambda: (0, 0)),
                  pl.BlockSpec((1, 16), lambda: (0, 0)),
                  pl.BlockSpec((16, 128), lambda: (0, 0)),
                  pl.BlockSpec((1, 128), lambda: (0, 0)),
                  pl.BlockSpec((N_NODES, 128), lambda: (0, 0))],
        out_shape=jax.ShapeDtypeStruct((N_NODES, 128), jnp.float32),
    )(q, cntc, bn_gamma.reshape(1, 16), bn_beta.reshape(1, 16),
      lin2_w, lin2_b.reshape(1, 128), skip)


def kernel(edge_vec, node_feature, edge_index, edge_feature, lin1_w, lin1_b,
           fc1_w1, fc1_b1, fc1_w2, fc1_b2, fc2_w1, fc2_b1, fc2_w2, fc2_b2,
           bn_gamma, bn_beta, lin2_w, lin2_b):
    A1 = fc1_w2[:, _COL1].reshape(256, 24)
    B1 = fc1_b2[_COL1]
    A2 = fc2_w2[:, _COL2].reshape(384, 16)
    B2 = fc2_b2[_COL2]

    pe = IDX_PAD - N_EDGES
    src2d = jnp.pad(edge_index[0], (0, pe)).reshape(IDX_PAD // GRP, GRP)
    dst2d = jnp.pad(edge_index[1], (0, pe)).reshape(IDX_PAD // GRP, GRP)

    evT = edge_vec.T
    efT = edge_feature.T
    h = _tc_h(node_feature, lin1_w, lin1_b)
    hd = _sc_gather(h, dst2d, 16, gblk=5)
    tp1e = _tc_tp1(evT, efT, hd, fc1_w1, fc1_b1, A1, B1)
    p = _sc_scatter_add(tp1e, src2d, 64, gblk=2)
    out1, cntc = _tc_out1(p, h)
    od = _sc_gather(out1, dst2d, 48, gblk=5)
    tp2 = _tc_tp2(evT, efT, od, fc2_w1, fc2_b1, A2, B2)
    q = _sc_scatter_add(tp2, src2d, 16, gblk=5)
    return _tc_final(q, cntc, bn_gamma, bn_beta, lin2_w, lin2_b,
                     node_feature)
